# baseline (speedup 1.0000x reference)
import jax
import jax.numpy as jnp
from jax.experimental import pallas as pl

B = 4
N = 8192
NPOINT = 2048
R0, R1 = 0.2, 0.4
K0, K1 = 32, 32
CIN = 64
EXP = 4
C2 = CIN * 2


def _index_points(points, idx):
    return jax.vmap(lambda p, i: p[i])(points, idx)


def _fps(xyz, npoint):
    b, n, _ = xyz.shape

    def step(state, _):
        distance, farthest = state
        centroid = jnp.take_along_axis(xyz, farthest[:, None, None], axis=1)
        dist = jnp.sum((xyz - centroid) ** 2, axis=-1)
        distance = jnp.minimum(distance, dist)
        new_farthest = jnp.argmax(distance, axis=-1).astype(jnp.int32)
        return (distance, new_farthest), farthest

    init = (jnp.full((b, n), 1e10, dtype=xyz.dtype), jnp.zeros((b,), jnp.int32))
    _, idxs = jax.lax.scan(step, init, None, length=npoint)
    return idxs.T


def _query(radius, nsample, xyz, new_xyz):
    sqrdists = jnp.sum((new_xyz[:, :, None, :] - xyz[:, None, :, :]) ** 2, axis=-1)
    neg_d, idx = jax.lax.top_k(-sqrdists, nsample)
    d = -neg_d
    idx = jnp.where(d > radius * radius, idx[..., :1], idx)
    return idx


def _identity_pallas(x):
    def body(x_ref, o_ref):
        o_ref[...] = x_ref[...]

    return pl.pallas_call(
        body, out_shape=jax.ShapeDtypeStruct(x.shape, x.dtype))(x)


def kernel(points_coor, points_fea, sa_W, sa_b, la_W, la_b, pw_W1, pw_b1, pw_W2, pw_b2):
    pc = points_coor.transpose(0, 2, 1)
    pf = points_fea.transpose(0, 2, 1)
    fps_idx = _fps(jax.lax.stop_gradient(pc), NPOINT)
    new_coor = _index_points(pc, fps_idx)
    gi = _query(R0, K0, pc, new_coor)
    gc = (_index_points(pc, gi) - new_coor[:, :, None, :]) / R0
    gf = _index_points(pf, gi)
    g = jnp.concatenate([gf, gc], axis=-1)
    h = jax.nn.relu(jnp.einsum('bskc,dc->bskd', g, sa_W) + sa_b)
    fea = jnp.max(h, axis=2)
    identity = fea
    gi2 = _query(R1, K1, new_coor, new_coor)
    gc2 = (_index_points(new_coor, gi2) - new_coor[:, :, None, :]) / R1
    gf2 = _index_points(fea, gi2)
    g2 = jnp.concatenate([gf2, gc2], axis=-1)
    h2 = jax.nn.relu(jnp.einsum('bskc,dc->bskd', g2, la_W) + la_b)
    fea2 = jnp.max(h2, axis=2)
    p = jax.nn.relu(fea2 @ pw_W1.T + pw_b1)
    p = p @ pw_W2.T + pw_b2
    fea_out = jax.nn.relu(p + identity)
    fea_out = _identity_pallas(fea_out)
    return new_coor.transpose(0, 2, 1), fea_out.transpose(0, 2, 1)


# speedup vs baseline: 1.7535x; 1.7535x over previous
import functools

import jax
import jax.numpy as jnp
from jax import lax
from jax.experimental import pallas as pl
from jax.experimental.pallas import tpu as pltpu
from jax.experimental.pallas import tpu_sc as plsc

B = 4
N = 8192
S = 2048
R0, R1 = 0.2, 0.4
K0, K1 = 32, 32
CIN = 64
C2 = CIN * 2
C4 = C2 * 4

NC = 2
NS = 16
NW = NC * NS


def _fps_body(x_ref, y_ref, z_ref, ox_ref, oy_ref, oz_ref):
    X = x_ref[...]
    Y = y_ref[...]
    Z = z_ref[...]
    lin = (lax.broadcasted_iota(jnp.int32, (B, 64, 128), 1) * 128
           + lax.broadcasted_iota(jnp.int32, (B, 64, 128), 2))
    row8 = lax.broadcasted_iota(jnp.int32, (B, 8, 128), 1)
    lane = lax.broadcasted_iota(jnp.int32, (B, 8, 128), 2)
    BIG = jnp.int32(N)

    def pick(mask, V):
        m1 = jnp.max(jnp.where(mask, V, -jnp.inf), axis=2, keepdims=True)
        return jnp.max(m1, axis=1, keepdims=True)

    cx0 = X[:, 0:1, 0:1]
    cy0 = Y[:, 0:1, 0:1]
    cz0 = Z[:, 0:1, 0:1]
    D0 = jnp.full((B, 64, 128), 1e10, dtype=jnp.float32)
    acc0 = jnp.zeros((B, 8, 128), jnp.float32)

    def step(t, state):
        D, cx, cy, cz, ax, ay, az = state
        r = (t // 128) % 8
        l = t % 128
        cond = (row8 == r) & (lane == l)
        ax = jnp.where(cond, cx, ax)
        ay = jnp.where(cond, cy, ay)
        az = jnp.where(cond, cz, az)
        d = (X - cx) ** 2 + (Y - cy) ** 2 + (Z - cz) ** 2
        D = jnp.minimum(D, d)
        m1 = jnp.max(jnp.max(D, axis=2, keepdims=True), axis=1, keepdims=True)
        eq = D == m1
        sel = jnp.where(eq, lin, BIG)
        idx = jnp.min(jnp.min(sel, axis=2, keepdims=True), axis=1, keepdims=True)
        pickm = lin == idx
        ncx = pick(pickm, X)
        ncy = pick(pickm, Y)
        ncz = pick(pickm, Z)

        @pl.when(t % 1024 == 1023)
        def _():
            blk = t // 1024
            ox_ref[:, pl.ds(blk * 8, 8), :] = ax
            oy_ref[:, pl.ds(blk * 8, 8), :] = ay
            oz_ref[:, pl.ds(blk * 8, 8), :] = az

        return D, ncx, ncy, ncz, ax, ay, az

    lax.fori_loop(0, S, step, (D0, cx0, cy0, cz0, acc0, acc0, acc0))


def _fps_call(xb, yb, zb):
    o = jax.ShapeDtypeStruct((B, 16, 128), jnp.float32)
    return pl.pallas_call(_fps_body, out_shape=(o, o, o))(xb, yb, zb)


def _knn_body(qx_ref, qy_ref, qz_ref, px_ref, py_ref, pz_ref, o_ref, d_scr,
              *, n, k, r2, qt):
    b = pl.program_id(0)
    qx = qx_ref[0]
    qy = qy_ref[0]
    qz = qz_ref[0]
    nch = n // 128
    lane128 = lax.broadcasted_iota(jnp.int32, (qt, 128), 1)

    def build(c, _):
        px = px_ref[0, :, pl.ds(c * 128, 128)]
        py = py_ref[0, :, pl.ds(c * 128, 128)]
        pz = pz_ref[0, :, pl.ds(c * 128, 128)]
        d_scr[:, pl.ds(c * 128, 128)] = (
            (qx - px) ** 2 + (qy - py) ** 2 + (qz - pz) ** 2)
        return 0

    lax.fori_loop(0, nch, build, 0)

    INF = jnp.float32(jnp.inf)
    idx0 = None
    for j in range(k):
        def pa(c, m128):
            return jnp.minimum(m128, d_scr[:, pl.ds(c * 128, 128)])

        m128 = lax.fori_loop(0, nch, pa, jnp.full((qt, 128), INF))
        m = jnp.min(m128, axis=1, keepdims=True)

        def pb(c, iacc):
            d = d_scr[:, pl.ds(c * 128, 128)]
            eq = d == m
            gl = lane128 + c * 128
            iacc = jnp.minimum(
                iacc, jnp.min(jnp.where(eq, gl, n), axis=1, keepdims=True))
            d_scr[:, pl.ds(c * 128, 128)] = jnp.where(eq, INF, d)
            return iacc

        idx = lax.fori_loop(0, nch, pb, jnp.full((qt, 1), n, jnp.int32))
        if j == 0:
            idx0 = idx
        else:
            idx = jnp.where(m > r2, idx0, idx)
        o_ref[0, :, pl.ds(j, 1)] = idx + b * n


def _knn_call(qx, qy, qz, px, py, pz, n, r2, k=32, qt=128):
    qspec = pl.BlockSpec((1, qt, 1), lambda b, s: (b, s, 0))
    pspec = pl.BlockSpec((1, 1, n), lambda b, s: (b, 0, 0))
    return pl.pallas_call(
        functools.partial(_knn_body, n=n, k=k, r2=r2, qt=qt),
        grid=(B, S // qt),
        in_specs=[qspec, qspec, qspec, pspec, pspec, pspec],
        out_specs=pl.BlockSpec((1, qt, k), lambda b, s: (b, s, 0)),
        out_shape=jax.ShapeDtypeStruct((B, S, k), jnp.int32),
        scratch_shapes=[pltpu.VMEM((qt, n), jnp.float32)],
    )(qx, qy, qz, px, py, pz)


def _dot(a, b):
    return lax.dot_general(a, b, (((1,), (0,)), ((), ())),
                           preferred_element_type=jnp.float32)


def _a1_body(g_ref, w_ref, o_ref):
    o_ref[0] = _dot(g_ref[0], w_ref[...])


def _a1_call(G1t, Wt):
    nb = 512
    return pl.pallas_call(
        _a1_body,
        grid=(B, N // nb),
        in_specs=[pl.BlockSpec((1, nb, CIN + 3), lambda b, i: (b, i, 0)),
                  pl.BlockSpec((CIN + 3, C2), lambda b, i: (0, 0))],
        out_specs=pl.BlockSpec((1, nb, C2), lambda b, i: (b, i, 0)),
        out_shape=jax.ShapeDtypeStruct((B, N, C2), jnp.float32),
    )(G1t, Wt)


def _mid_body(mg_ref, q_ref, wc1_ref, sab_ref, wf2_ref, wc2_ref,
              fea_ref, a2_ref, qc2_ref):
    q = q_ref[...]
    qc1 = _dot(q, wc1_ref[...])
    fea = jax.nn.relu(mg_ref[...] + sab_ref[...] - qc1)
    qc2 = _dot(q, wc2_ref[...])
    fea_ref[...] = fea
    a2_ref[...] = _dot(fea, wf2_ref[...]) + qc2
    qc2_ref[...] = qc2


def _mid_call(mg1, q3, wc1t, sab, wf2t, wc2t):
    nb = 512
    BS = B * S
    spec = pl.BlockSpec((nb, C2), lambda i: (i, 0))
    o = jax.ShapeDtypeStruct((BS, C2), jnp.float32)
    return pl.pallas_call(
        _mid_body,
        grid=(BS // nb,),
        in_specs=[spec,
                  pl.BlockSpec((nb, 3), lambda i: (i, 0)),
                  pl.BlockSpec((3, C2), lambda i: (0, 0)),
                  pl.BlockSpec((1, C2), lambda i: (0, 0)),
                  pl.BlockSpec((C2, C2), lambda i: (0, 0)),
                  pl.BlockSpec((3, C2), lambda i: (0, 0))],
        out_specs=[spec, spec, spec],
        out_shape=(o, o, o),
    )(mg1, q3, wc1t, sab, wf2t, wc2t)


def _final_body(mg_ref, qc2_ref, fea_ref, lab_ref, w1_ref, b1_ref, w2_ref,
                b2_ref, o_ref):
    fea2 = jax.nn.relu(mg_ref[...] + lab_ref[...] - qc2_ref[...])
    p = jax.nn.relu(_dot(fea2, w1_ref[...]) + b1_ref[...])
    p2 = _dot(p, w2_ref[...]) + b2_ref[...]
    o_ref[...] = jax.nn.relu(p2 + fea_ref[...])


def _final_call(mg2, qc2, fea, lab, w1t, b1, w2t, b2):
    nb = 512
    BS = B * S
    spec = pl.BlockSpec((nb, C2), lambda i: (i, 0))
    return pl.pallas_call(
        _final_body,
        grid=(BS // nb,),
        in_specs=[spec, spec, spec,
                  pl.BlockSpec((1, C2), lambda i: (0, 0)),
                  pl.BlockSpec((C2, C4), lambda i: (0, 0)),
                  pl.BlockSpec((1, C4), lambda i: (0, 0)),
                  pl.BlockSpec((C4, C2), lambda i: (0, 0)),
                  pl.BlockSpec((1, C2), lambda i: (0, 0))],
        out_specs=spec,
        out_shape=jax.ShapeDtypeStruct((BS, C2), jnp.float32),
    )(mg2, qc2, fea, lab, w1t, b1, w2t, b2)


def _gather_max(table, idx2d, k):
    Q = idx2d.shape[0] * 128 // k
    qpw = Q // NW
    QB = 128 // k
    nchunks = qpw // QB
    mesh = plsc.VectorSubcoreMesh(core_axis_name="c", subcore_axis_name="s",
                                  num_cores=NC, num_subcores=NS)

    @functools.partial(
        pl.kernel,
        out_type=jax.ShapeDtypeStruct((Q, C2), jnp.float32),
        mesh=mesh,
        scratch_types=[
            pltpu.VMEM((nchunks, 128), jnp.int32),
            pltpu.VMEM((128, C2), jnp.float32),
            pltpu.VMEM((128, C2), jnp.float32),
            pltpu.VMEM((qpw, C2), jnp.float32),
            pltpu.SemaphoreType.DMA,
            pltpu.SemaphoreType.DMA,
        ],
    )
    def gk(idx_hbm, table_hbm, out_hbm, idx_v, rows0, rows1, out_v, sem0, sem1):
        wid = lax.axis_index("s") * NC + lax.axis_index("c")
        pltpu.sync_copy(idx_hbm.at[pl.ds(wid * nchunks, nchunks)], idx_v)

        def start(i, rbuf, sem):
            pltpu.async_copy(table_hbm.at[idx_v.at[i]], rbuf, sem)

        def wait(i, rbuf, sem):
            pltpu.make_async_copy(table_hbm.at[idx_v.at[i]], rbuf, sem).wait()

        def compute(i, rbuf):
            def per_q(q, _):
                base = q * k
                acc = tuple(rbuf[base, c * 16:(c + 1) * 16] for c in range(8))

                def red(j, a):
                    return tuple(
                        jnp.maximum(a[c], rbuf[base + j, c * 16:(c + 1) * 16])
                        for c in range(8))

                acc = lax.fori_loop(1, k, red, acc)
                for c in range(8):
                    out_v[i * QB + q, c * 16:(c + 1) * 16] = acc[c]
                return 0

            lax.fori_loop(0, QB, per_q, 0)

        start(0, rows0, sem0)

        def pair(p, _):
            i0 = p * 2
            start(i0 + 1, rows1, sem1)
            wait(i0, rows0, sem0)
            compute(i0, rows0)

            @pl.when(i0 + 2 < nchunks)
            def _():
                start(i0 + 2, rows0, sem0)

            wait(i0 + 1, rows1, sem1)
            compute(i0 + 1, rows1)
            return 0

        lax.fori_loop(0, nchunks // 2, pair, 0)
        pltpu.sync_copy(out_v, out_hbm.at[pl.ds(wid * qpw, qpw)])

    return gk(idx2d, table)


def kernel(points_coor, points_fea, sa_W, sa_b, la_W, la_b,
           pw_W1, pw_b1, pw_W2, pw_b2):
    pc = points_coor
    x = pc[:, 0, :]
    y = pc[:, 1, :]
    z = pc[:, 2, :]
    ox, oy, oz = _fps_call(x.reshape(B, 64, 128), y.reshape(B, 64, 128),
                           z.reshape(B, 64, 128))
    qx = ox.reshape(B, S)
    qy = oy.reshape(B, S)
    qz = oz.reshape(B, S)

    idx1 = _knn_call(qx.reshape(B, S, 1), qy.reshape(B, S, 1),
                     qz.reshape(B, S, 1),
                     x.reshape(B, 1, N), y.reshape(B, 1, N),
                     z.reshape(B, 1, N), n=N, r2=R0 * R0, k=K0)
    G1t = jnp.concatenate([points_fea, pc / R0], axis=1).transpose(0, 2, 1)
    A1 = _a1_call(G1t, sa_W.T).reshape(B * N, C2)
    mg1 = _gather_max(A1, idx1.reshape(-1, 128), K0)

    q3 = jnp.stack([qx, qy, qz], axis=-1).reshape(B * S, 3)
    fea, A2, QC2 = _mid_call(mg1, q3, sa_W[:, CIN:].T / R0,
                             sa_b.reshape(1, C2), la_W[:, :C2].T,
                             la_W[:, C2:].T / R1)

    idx2 = _knn_call(qx.reshape(B, S, 1), qy.reshape(B, S, 1),
                     qz.reshape(B, S, 1),
                     qx.reshape(B, 1, S), qy.reshape(B, 1, S),
                     qz.reshape(B, 1, S), n=S, r2=R1 * R1, k=K1)
    mg2 = _gather_max(A2, idx2.reshape(-1, 128), K1)

    out = _final_call(mg2, QC2, fea, la_b.reshape(1, C2), pw_W1.T,
                      pw_b1.reshape(1, C4), pw_W2.T, pw_b2.reshape(1, C2))
    new_coor = jnp.stack([qx, qy, qz], axis=1)
    return new_coor, out.reshape(B, S, C2).transpose(0, 2, 1)


# speedup vs baseline: 8.1079x; 4.6238x over previous
import functools

import jax
import jax.numpy as jnp
from jax import lax
from jax.experimental import pallas as pl
from jax.experimental.pallas import tpu as pltpu
from jax.experimental.pallas import tpu_sc as plsc

B = 4
N = 8192
S = 2048
R0, R1 = 0.2, 0.4
K0, K1 = 32, 32
CIN = 64
C2 = CIN * 2
C4 = C2 * 4

NC = 2
NS = 16
NW = NC * NS


def _fps_body(x_ref, y_ref, z_ref, ox_ref, oy_ref, oz_ref):
    X = x_ref[...]
    Y = y_ref[...]
    Z = z_ref[...]
    lin = (lax.broadcasted_iota(jnp.int32, (B, 64, 128), 1) * 128
           + lax.broadcasted_iota(jnp.int32, (B, 64, 128), 2))
    row8 = lax.broadcasted_iota(jnp.int32, (B, 8, 128), 1)
    lane = lax.broadcasted_iota(jnp.int32, (B, 8, 128), 2)
    BIG = jnp.int32(N)

    def pick(mask, V):
        m1 = jnp.max(jnp.where(mask, V, -jnp.inf), axis=2, keepdims=True)
        return jnp.max(m1, axis=1, keepdims=True)

    cx0 = X[:, 0:1, 0:1]
    cy0 = Y[:, 0:1, 0:1]
    cz0 = Z[:, 0:1, 0:1]
    D0 = jnp.full((B, 64, 128), 1e10, dtype=jnp.float32)
    acc0 = jnp.zeros((B, 8, 128), jnp.float32)

    def step(t, state):
        D, cx, cy, cz, ax, ay, az = state
        r = (t // 128) % 8
        l = t % 128
        cond = (row8 == r) & (lane == l)
        ax = jnp.where(cond, cx, ax)
        ay = jnp.where(cond, cy, ay)
        az = jnp.where(cond, cz, az)
        d = (X - cx) ** 2 + (Y - cy) ** 2 + (Z - cz) ** 2
        D = jnp.minimum(D, d)
        m1 = jnp.max(jnp.max(D, axis=2, keepdims=True), axis=1, keepdims=True)
        eq = D == m1
        sel = jnp.where(eq, lin, BIG)
        idx = jnp.min(jnp.min(sel, axis=2, keepdims=True), axis=1, keepdims=True)
        pickm = lin == idx
        ncx = pick(pickm, X)
        ncy = pick(pickm, Y)
        ncz = pick(pickm, Z)

        @pl.when(t % 1024 == 1023)
        def _():
            blk = t // 1024
            ox_ref[:, pl.ds(blk * 8, 8), :] = ax
            oy_ref[:, pl.ds(blk * 8, 8), :] = ay
            oz_ref[:, pl.ds(blk * 8, 8), :] = az

        return D, ncx, ncy, ncz, ax, ay, az

    lax.fori_loop(0, S, step, (D0, cx0, cy0, cz0, acc0, acc0, acc0))


def _fps_call(xb, yb, zb):
    o = jax.ShapeDtypeStruct((B, 16, 128), jnp.float32)
    return pl.pallas_call(_fps_body, out_shape=(o, o, o))(xb, yb, zb)


def _knn_body(qx_ref, qy_ref, qz_ref, px_ref, py_ref, pz_ref, o_ref, d_scr,
              *, n, k, r2, qt):
    b = pl.program_id(0)
    qx = qx_ref[0]
    qy = qy_ref[0]
    qz = qz_ref[0]
    CW = 256
    nch = n // CW
    INF = jnp.float32(jnp.inf)
    BIGI = jnp.int32(n)
    lane = lax.broadcasted_iota(jnp.int32, (qt, CW), 1)

    def build(c, nm):
        px = px_ref[0, :, pl.ds(c * CW, CW)]
        py = py_ref[0, :, pl.ds(c * CW, CW)]
        pz = pz_ref[0, :, pl.ds(c * CW, CW)]
        d = (qx - px) ** 2 + (qy - py) ** 2 + (qz - pz) ** 2
        d_scr[:, pl.ds(c * CW, CW)] = d
        return jnp.minimum(nm, d)

    nm = lax.fori_loop(0, nch, build, jnp.full((qt, CW), INF))
    m = jnp.min(nm, axis=1, keepdims=True)

    idx0 = None
    for j in range(1, k + 1):
        last = j == k

        def body(c, carry, m=m, last=last):
            nm, selm = carry
            dd = d_scr[:, pl.ds(c * CW, CW)]
            gl = lane + c * CW
            selm = jnp.minimum(selm, jnp.where(dd == m, gl, BIGI))
            if not last:
                nm = jnp.minimum(nm, jnp.where(dd > m, dd, INF))
            return nm, selm

        nm, selm = lax.fori_loop(
            0, nch, body,
            (jnp.full((qt, CW), INF), jnp.full((qt, CW), BIGI)))
        idx = jnp.min(selm, axis=1, keepdims=True)
        if j == 1:
            idx0 = idx
            oidx = idx
        else:
            oidx = jnp.where(m > r2, idx0, idx)
        o_ref[0, :, pl.ds(j - 1, 1)] = oidx + b * n
        if not last:
            m = jnp.min(nm, axis=1, keepdims=True)


def _knn_call(qx, qy, qz, px, py, pz, n, r2, k=32, qt=128):
    qspec = pl.BlockSpec((1, qt, 1), lambda b, s: (b, s, 0))
    pspec = pl.BlockSpec((1, 1, n), lambda b, s: (b, 0, 0))
    return pl.pallas_call(
        functools.partial(_knn_body, n=n, k=k, r2=r2, qt=qt),
        grid=(B, S // qt),
        in_specs=[qspec, qspec, qspec, pspec, pspec, pspec],
        out_specs=pl.BlockSpec((1, qt, k), lambda b, s: (b, s, 0)),
        out_shape=jax.ShapeDtypeStruct((B, S, k), jnp.int32),
        scratch_shapes=[pltpu.VMEM((qt, n), jnp.float32)],
    )(qx, qy, qz, px, py, pz)


def _dot(a, b):
    return lax.dot_general(a, b, (((1,), (0,)), ((), ())),
                           preferred_element_type=jnp.float32)


def _a1_body(g_ref, w_ref, o_ref):
    o_ref[0] = _dot(g_ref[0], w_ref[...])


def _a1_call(G1t, Wt):
    nb = 512
    return pl.pallas_call(
        _a1_body,
        grid=(B, N // nb),
        in_specs=[pl.BlockSpec((1, nb, CIN + 3), lambda b, i: (b, i, 0)),
                  pl.BlockSpec((CIN + 3, C2), lambda b, i: (0, 0))],
        out_specs=pl.BlockSpec((1, nb, C2), lambda b, i: (b, i, 0)),
        out_shape=jax.ShapeDtypeStruct((B, N, C2), jnp.float32),
    )(G1t, Wt)


def _mid_body(mg_ref, q_ref, wc1_ref, sab_ref, wf2_ref, wc2_ref,
              fea_ref, a2_ref, qc2_ref):
    q = q_ref[...]
    qc1 = _dot(q, wc1_ref[...])
    fea = jax.nn.relu(mg_ref[...] + sab_ref[...] - qc1)
    qc2 = _dot(q, wc2_ref[...])
    fea_ref[...] = fea
    a2_ref[...] = _dot(fea, wf2_ref[...]) + qc2
    qc2_ref[...] = qc2


def _mid_call(mg1, q3, wc1t, sab, wf2t, wc2t):
    nb = 512
    BS = B * S
    spec = pl.BlockSpec((nb, C2), lambda i: (i, 0))
    o = jax.ShapeDtypeStruct((BS, C2), jnp.float32)
    return pl.pallas_call(
        _mid_body,
        grid=(BS // nb,),
        in_specs=[spec,
                  pl.BlockSpec((nb, 3), lambda i: (i, 0)),
                  pl.BlockSpec((3, C2), lambda i: (0, 0)),
                  pl.BlockSpec((1, C2), lambda i: (0, 0)),
                  pl.BlockSpec((C2, C2), lambda i: (0, 0)),
                  pl.BlockSpec((3, C2), lambda i: (0, 0))],
        out_specs=[spec, spec, spec],
        out_shape=(o, o, o),
    )(mg1, q3, wc1t, sab, wf2t, wc2t)


def _final_body(mg_ref, qc2_ref, fea_ref, lab_ref, w1_ref, b1_ref, w2_ref,
                b2_ref, o_ref):
    fea2 = jax.nn.relu(mg_ref[...] + lab_ref[...] - qc2_ref[...])
    p = jax.nn.relu(_dot(fea2, w1_ref[...]) + b1_ref[...])
    p2 = _dot(p, w2_ref[...]) + b2_ref[...]
    o_ref[...] = jax.nn.relu(p2 + fea_ref[...])


def _final_call(mg2, qc2, fea, lab, w1t, b1, w2t, b2):
    nb = 512
    BS = B * S
    spec = pl.BlockSpec((nb, C2), lambda i: (i, 0))
    return pl.pallas_call(
        _final_body,
        grid=(BS // nb,),
        in_specs=[spec, spec, spec,
                  pl.BlockSpec((1, C2), lambda i: (0, 0)),
                  pl.BlockSpec((C2, C4), lambda i: (0, 0)),
                  pl.BlockSpec((1, C4), lambda i: (0, 0)),
                  pl.BlockSpec((C4, C2), lambda i: (0, 0)),
                  pl.BlockSpec((1, C2), lambda i: (0, 0))],
        out_specs=spec,
        out_shape=jax.ShapeDtypeStruct((BS, C2), jnp.float32),
    )(mg2, qc2, fea, lab, w1t, b1, w2t, b2)


def _gather_max(table, idx2d, k):
    Q = idx2d.shape[0] * 128 // k
    qpw = Q // NW
    QB = 128 // k
    nchunks = qpw // QB
    mesh = plsc.VectorSubcoreMesh(core_axis_name="c", subcore_axis_name="s",
                                  num_cores=NC, num_subcores=NS)

    @functools.partial(
        pl.kernel,
        out_type=jax.ShapeDtypeStruct((Q, C2), jnp.float32),
        mesh=mesh,
        scratch_types=[
            pltpu.VMEM((nchunks, 128), jnp.int32),
            pltpu.VMEM((128, C2), jnp.float32),
            pltpu.VMEM((128, C2), jnp.float32),
            pltpu.VMEM((qpw, C2), jnp.float32),
            pltpu.SemaphoreType.DMA,
            pltpu.SemaphoreType.DMA,
        ],
    )
    def gk(idx_hbm, table_hbm, out_hbm, idx_v, rows0, rows1, out_v, sem0, sem1):
        wid = lax.axis_index("s") * NC + lax.axis_index("c")
        pltpu.sync_copy(idx_hbm.at[pl.ds(wid * nchunks, nchunks)], idx_v)

        def start(i, rbuf, sem):
            pltpu.async_copy(table_hbm.at[idx_v.at[i]], rbuf, sem)

        def wait(i, rbuf, sem):
            pltpu.make_async_copy(table_hbm.at[idx_v.at[i]], rbuf, sem).wait()

        def compute(i, rbuf):
            def per_q(q, _):
                base = q * k
                acc = tuple(rbuf[base, c * 16:(c + 1) * 16] for c in range(8))

                def red(j, a):
                    return tuple(
                        jnp.maximum(a[c], rbuf[base + j, c * 16:(c + 1) * 16])
                        for c in range(8))

                acc = lax.fori_loop(1, k, red, acc)
                for c in range(8):
                    out_v[i * QB + q, c * 16:(c + 1) * 16] = acc[c]
                return 0

            lax.fori_loop(0, QB, per_q, 0)

        start(0, rows0, sem0)

        def pair(p, _):
            i0 = p * 2
            start(i0 + 1, rows1, sem1)
            wait(i0, rows0, sem0)
            compute(i0, rows0)

            @pl.when(i0 + 2 < nchunks)
            def _():
                start(i0 + 2, rows0, sem0)

            wait(i0 + 1, rows1, sem1)
            compute(i0 + 1, rows1)
            return 0

        lax.fori_loop(0, nchunks // 2, pair, 0)
        pltpu.sync_copy(out_v, out_hbm.at[pl.ds(wid * qpw, qpw)])

    return gk(idx2d, table)


def kernel(points_coor, points_fea, sa_W, sa_b, la_W, la_b,
           pw_W1, pw_b1, pw_W2, pw_b2):
    pc = points_coor
    x = pc[:, 0, :]
    y = pc[:, 1, :]
    z = pc[:, 2, :]
    ox, oy, oz = _fps_call(x.reshape(B, 64, 128), y.reshape(B, 64, 128),
                           z.reshape(B, 64, 128))
    qx = ox.reshape(B, S)
    qy = oy.reshape(B, S)
    qz = oz.reshape(B, S)

    idx1 = _knn_call(qx.reshape(B, S, 1), qy.reshape(B, S, 1),
                     qz.reshape(B, S, 1),
                     x.reshape(B, 1, N), y.reshape(B, 1, N),
                     z.reshape(B, 1, N), n=N, r2=R0 * R0, k=K0)
    G1t = jnp.concatenate([points_fea, pc / R0], axis=1).transpose(0, 2, 1)
    A1 = _a1_call(G1t, sa_W.T).reshape(B * N, C2)
    mg1 = _gather_max(A1, idx1.reshape(-1, 128), K0)

    q3 = jnp.stack([qx, qy, qz], axis=-1).reshape(B * S, 3)
    fea, A2, QC2 = _mid_call(mg1, q3, sa_W[:, CIN:].T / R0,
                             sa_b.reshape(1, C2), la_W[:, :C2].T,
                             la_W[:, C2:].T / R1)

    idx2 = _knn_call(qx.reshape(B, S, 1), qy.reshape(B, S, 1),
                     qz.reshape(B, S, 1),
                     qx.reshape(B, 1, S), qy.reshape(B, 1, S),
                     qz.reshape(B, 1, S), n=S, r2=R1 * R1, k=K1)
    mg2 = _gather_max(A2, idx2.reshape(-1, 128), K1)

    out = _final_call(mg2, QC2, fea, la_b.reshape(1, C2), pw_W1.T,
                      pw_b1.reshape(1, C4), pw_W2.T, pw_b2.reshape(1, C2))
    new_coor = jnp.stack([qx, qy, qz], axis=1)
    return new_coor, out.reshape(B, S, C2).transpose(0, 2, 1)


# speedup vs baseline: 8.9231x; 1.1005x over previous
import functools

import jax
import jax.numpy as jnp
from jax import lax
from jax.experimental import pallas as pl
from jax.experimental.pallas import tpu as pltpu
from jax.experimental.pallas import tpu_sc as plsc

B = 4
N = 8192
S = 2048
R0, R1 = 0.2, 0.4
K0, K1 = 32, 32
CIN = 64
C2 = CIN * 2
C4 = C2 * 4

NC = 2
NS = 16
NW = NC * NS


def _fps_body(x_ref, y_ref, z_ref, ox_ref, oy_ref, oz_ref):
    X = x_ref[...]
    Y = y_ref[...]
    Z = z_ref[...]
    lin = (lax.broadcasted_iota(jnp.int32, (B, 64, 128), 1) * 128
           + lax.broadcasted_iota(jnp.int32, (B, 64, 128), 2))
    row8 = lax.broadcasted_iota(jnp.int32, (B, 8, 128), 1)
    lane = lax.broadcasted_iota(jnp.int32, (B, 8, 128), 2)
    BIG = jnp.int32(N)

    def pick(mask, V):
        m1 = jnp.max(jnp.where(mask, V, -jnp.inf), axis=2, keepdims=True)
        return jnp.max(m1, axis=1, keepdims=True)

    cx0 = X[:, 0:1, 0:1]
    cy0 = Y[:, 0:1, 0:1]
    cz0 = Z[:, 0:1, 0:1]
    D0 = jnp.full((B, 64, 128), 1e10, dtype=jnp.float32)
    acc0 = jnp.zeros((B, 8, 128), jnp.float32)

    def step(t, state):
        D, cx, cy, cz, ax, ay, az = state
        r = (t // 128) % 8
        l = t % 128
        cond = (row8 == r) & (lane == l)
        ax = jnp.where(cond, cx, ax)
        ay = jnp.where(cond, cy, ay)
        az = jnp.where(cond, cz, az)
        d = (X - cx) ** 2 + (Y - cy) ** 2 + (Z - cz) ** 2
        D = jnp.minimum(D, d)
        m1 = jnp.max(jnp.max(D, axis=2, keepdims=True), axis=1, keepdims=True)
        eq = D == m1
        sel = jnp.where(eq, lin, BIG)
        idx = jnp.min(jnp.min(sel, axis=2, keepdims=True), axis=1, keepdims=True)
        pickm = lin == idx
        ncx = pick(pickm, X)
        ncy = pick(pickm, Y)
        ncz = pick(pickm, Z)

        @pl.when(t % 1024 == 1023)
        def _():
            blk = t // 1024
            ox_ref[:, pl.ds(blk * 8, 8), :] = ax
            oy_ref[:, pl.ds(blk * 8, 8), :] = ay
            oz_ref[:, pl.ds(blk * 8, 8), :] = az

        return D, ncx, ncy, ncz, ax, ay, az

    lax.fori_loop(0, S, step, (D0, cx0, cy0, cz0, acc0, acc0, acc0))


def _fps_call(xb, yb, zb):
    o = jax.ShapeDtypeStruct((B, 16, 128), jnp.float32)
    return pl.pallas_call(_fps_body, out_shape=(o, o, o))(xb, yb, zb)


def _knn_body(qx_ref, qy_ref, qz_ref, px_ref, py_ref, pz_ref, o_ref, d_scr,
              *, n, k, r2, qt):
    b = pl.program_id(0)
    qx = qx_ref[0]
    qy = qy_ref[0]
    qz = qz_ref[0]
    CW = 128
    nch = n // CW
    INF = jnp.float32(jnp.inf)
    BIGF = jnp.float32(n)
    lane = lax.broadcasted_iota(jnp.int32, (qt, CW), 1).astype(jnp.float32)
    nm0 = jnp.full((qt, CW), INF)
    gi0 = jnp.full((qt, CW), BIGF)

    def build(c, carry):
        nm, gi = carry
        px = px_ref[0, :, pl.ds(c * CW, CW)]
        py = py_ref[0, :, pl.ds(c * CW, CW)]
        pz = pz_ref[0, :, pl.ds(c * CW, CW)]
        d = (qx - px) ** 2 + (qy - py) ** 2 + (qz - pz) ** 2
        d_scr[:, pl.ds(c * CW, CW)] = d
        gl = lane + (c * CW).astype(jnp.float32)
        better = d < nm
        nm = jnp.where(better, d, nm)
        gi = jnp.where(better, gl, gi)
        return nm, gi

    nm, gi = lax.fori_loop(0, nch, build, (nm0, gi0))

    def epilogue(nm, gi):
        m = jnp.min(nm, axis=1, keepdims=True)
        idxf = jnp.min(jnp.where(nm == m, gi, BIGF), axis=1, keepdims=True)
        return m, idxf

    m, idx0 = epilogue(nm, gi)
    o_ref[0, :, pl.ds(0, 1)] = idx0.astype(jnp.int32) + b * n

    for j in range(1, k):
        def body(c, carry, m=m):
            nm, gi = carry
            dd = d_scr[:, pl.ds(c * CW, CW)]
            gl = lane + (c * CW).astype(jnp.float32)
            cand = jnp.where(dd > m, dd, INF)
            better = cand < nm
            nm = jnp.where(better, cand, nm)
            gi = jnp.where(better, gl, gi)
            return nm, gi

        nm, gi = lax.fori_loop(0, nch, body, (nm0, gi0))
        m, idxf = epilogue(nm, gi)
        oidx = jnp.where(m > r2, idx0, idxf)
        o_ref[0, :, pl.ds(j, 1)] = oidx.astype(jnp.int32) + b * n


def _knn_call(qx, qy, qz, px, py, pz, n, r2, k=32, qt=128):
    qspec = pl.BlockSpec((1, qt, 1), lambda b, s: (b, s, 0))
    pspec = pl.BlockSpec((1, 1, n), lambda b, s: (b, 0, 0))
    return pl.pallas_call(
        functools.partial(_knn_body, n=n, k=k, r2=r2, qt=qt),
        grid=(B, S // qt),
        in_specs=[qspec, qspec, qspec, pspec, pspec, pspec],
        out_specs=pl.BlockSpec((1, qt, k), lambda b, s: (b, s, 0)),
        out_shape=jax.ShapeDtypeStruct((B, S, k), jnp.int32),
        scratch_shapes=[pltpu.VMEM((qt, n), jnp.float32)],
    )(qx, qy, qz, px, py, pz)


def _dot(a, b):
    return lax.dot_general(a, b, (((1,), (0,)), ((), ())),
                           preferred_element_type=jnp.float32)


def _a1_body(g_ref, w_ref, o_ref):
    o_ref[0] = _dot(g_ref[0], w_ref[...])


def _a1_call(G1t, Wt):
    nb = 512
    return pl.pallas_call(
        _a1_body,
        grid=(B, N // nb),
        in_specs=[pl.BlockSpec((1, nb, CIN + 3), lambda b, i: (b, i, 0)),
                  pl.BlockSpec((CIN + 3, C2), lambda b, i: (0, 0))],
        out_specs=pl.BlockSpec((1, nb, C2), lambda b, i: (b, i, 0)),
        out_shape=jax.ShapeDtypeStruct((B, N, C2), jnp.float32),
    )(G1t, Wt)


def _mid_body(mg_ref, q_ref, wc1_ref, sab_ref, wf2_ref, wc2_ref,
              fea_ref, a2_ref, qc2_ref):
    q = q_ref[...]
    qc1 = _dot(q, wc1_ref[...])
    fea = jax.nn.relu(mg_ref[...] + sab_ref[...] - qc1)
    qc2 = _dot(q, wc2_ref[...])
    fea_ref[...] = fea
    a2_ref[...] = _dot(fea, wf2_ref[...]) + qc2
    qc2_ref[...] = qc2


def _mid_call(mg1, q3, wc1t, sab, wf2t, wc2t):
    nb = 512
    BS = B * S
    spec = pl.BlockSpec((nb, C2), lambda i: (i, 0))
    o = jax.ShapeDtypeStruct((BS, C2), jnp.float32)
    return pl.pallas_call(
        _mid_body,
        grid=(BS // nb,),
        in_specs=[spec,
                  pl.BlockSpec((nb, 3), lambda i: (i, 0)),
                  pl.BlockSpec((3, C2), lambda i: (0, 0)),
                  pl.BlockSpec((1, C2), lambda i: (0, 0)),
                  pl.BlockSpec((C2, C2), lambda i: (0, 0)),
                  pl.BlockSpec((3, C2), lambda i: (0, 0))],
        out_specs=[spec, spec, spec],
        out_shape=(o, o, o),
    )(mg1, q3, wc1t, sab, wf2t, wc2t)


def _final_body(mg_ref, qc2_ref, fea_ref, lab_ref, w1_ref, b1_ref, w2_ref,
                b2_ref, o_ref):
    fea2 = jax.nn.relu(mg_ref[...] + lab_ref[...] - qc2_ref[...])
    p = jax.nn.relu(_dot(fea2, w1_ref[...]) + b1_ref[...])
    p2 = _dot(p, w2_ref[...]) + b2_ref[...]
    o_ref[...] = jax.nn.relu(p2 + fea_ref[...])


def _final_call(mg2, qc2, fea, lab, w1t, b1, w2t, b2):
    nb = 512
    BS = B * S
    spec = pl.BlockSpec((nb, C2), lambda i: (i, 0))
    return pl.pallas_call(
        _final_body,
        grid=(BS // nb,),
        in_specs=[spec, spec, spec,
                  pl.BlockSpec((1, C2), lambda i: (0, 0)),
                  pl.BlockSpec((C2, C4), lambda i: (0, 0)),
                  pl.BlockSpec((1, C4), lambda i: (0, 0)),
                  pl.BlockSpec((C4, C2), lambda i: (0, 0)),
                  pl.BlockSpec((1, C2), lambda i: (0, 0))],
        out_specs=spec,
        out_shape=jax.ShapeDtypeStruct((BS, C2), jnp.float32),
    )(mg2, qc2, fea, lab, w1t, b1, w2t, b2)


def _gather_max(table, idx2d, k):
    Q = idx2d.shape[0] * 128 // k
    qpw = Q // NW
    QB = 128 // k
    nchunks = qpw // QB
    mesh = plsc.VectorSubcoreMesh(core_axis_name="c", subcore_axis_name="s",
                                  num_cores=NC, num_subcores=NS)

    @functools.partial(
        pl.kernel,
        out_type=jax.ShapeDtypeStruct((Q, C2), jnp.float32),
        mesh=mesh,
        scratch_types=[
            pltpu.VMEM((nchunks, 128), jnp.int32),
            pltpu.VMEM((128, C2), jnp.float32),
            pltpu.VMEM((128, C2), jnp.float32),
            pltpu.VMEM((qpw, C2), jnp.float32),
            pltpu.SemaphoreType.DMA,
            pltpu.SemaphoreType.DMA,
        ],
    )
    def gk(idx_hbm, table_hbm, out_hbm, idx_v, rows0, rows1, out_v, sem0, sem1):
        wid = lax.axis_index("s") * NC + lax.axis_index("c")
        pltpu.sync_copy(idx_hbm.at[pl.ds(wid * nchunks, nchunks)], idx_v)

        def start(i, rbuf, sem):
            pltpu.async_copy(table_hbm.at[idx_v.at[i]], rbuf, sem)

        def wait(i, rbuf, sem):
            pltpu.make_async_copy(table_hbm.at[idx_v.at[i]], rbuf, sem).wait()

        def compute(i, rbuf):
            def per_q(q, _):
                base = q * k
                acc = tuple(rbuf[base, c * 16:(c + 1) * 16] for c in range(8))

                def red(j, a):
                    return tuple(
                        jnp.maximum(a[c], rbuf[base + j, c * 16:(c + 1) * 16])
                        for c in range(8))

                acc = lax.fori_loop(1, k, red, acc)
                for c in range(8):
                    out_v[i * QB + q, c * 16:(c + 1) * 16] = acc[c]
                return 0

            lax.fori_loop(0, QB, per_q, 0)

        start(0, rows0, sem0)

        def pair(p, _):
            i0 = p * 2
            start(i0 + 1, rows1, sem1)
            wait(i0, rows0, sem0)
            compute(i0, rows0)

            @pl.when(i0 + 2 < nchunks)
            def _():
                start(i0 + 2, rows0, sem0)

            wait(i0 + 1, rows1, sem1)
            compute(i0 + 1, rows1)
            return 0

        lax.fori_loop(0, nchunks // 2, pair, 0)
        pltpu.sync_copy(out_v, out_hbm.at[pl.ds(wid * qpw, qpw)])

    return gk(idx2d, table)


def kernel(points_coor, points_fea, sa_W, sa_b, la_W, la_b,
           pw_W1, pw_b1, pw_W2, pw_b2):
    pc = points_coor
    x = pc[:, 0, :]
    y = pc[:, 1, :]
    z = pc[:, 2, :]
    ox, oy, oz = _fps_call(x.reshape(B, 64, 128), y.reshape(B, 64, 128),
                           z.reshape(B, 64, 128))
    qx = ox.reshape(B, S)
    qy = oy.reshape(B, S)
    qz = oz.reshape(B, S)

    idx1 = _knn_call(qx.reshape(B, S, 1), qy.reshape(B, S, 1),
                     qz.reshape(B, S, 1),
                     x.reshape(B, 1, N), y.reshape(B, 1, N),
                     z.reshape(B, 1, N), n=N, r2=R0 * R0, k=K0)
    G1t = jnp.concatenate([points_fea, pc / R0], axis=1).transpose(0, 2, 1)
    A1 = _a1_call(G1t, sa_W.T).reshape(B * N, C2)
    mg1 = _gather_max(A1, idx1.reshape(-1, 128), K0)

    q3 = jnp.stack([qx, qy, qz], axis=-1).reshape(B * S, 3)
    fea, A2, QC2 = _mid_call(mg1, q3, sa_W[:, CIN:].T / R0,
                             sa_b.reshape(1, C2), la_W[:, :C2].T,
                             la_W[:, C2:].T / R1)

    idx2 = _knn_call(qx.reshape(B, S, 1), qy.reshape(B, S, 1),
                     qz.reshape(B, S, 1),
                     qx.reshape(B, 1, S), qy.reshape(B, 1, S),
                     qz.reshape(B, 1, S), n=S, r2=R1 * R1, k=K1)
    mg2 = _gather_max(A2, idx2.reshape(-1, 128), K1)

    out = _final_call(mg2, QC2, fea, la_b.reshape(1, C2), pw_W1.T,
                      pw_b1.reshape(1, C4), pw_W2.T, pw_b2.reshape(1, C2))
    new_coor = jnp.stack([qx, qy, qz], axis=1)
    return new_coor, out.reshape(B, S, C2).transpose(0, 2, 1)


# speedup vs baseline: 10.1115x; 1.1332x over previous
import functools

import jax
import jax.numpy as jnp
from jax import lax
from jax.experimental import pallas as pl
from jax.experimental.pallas import tpu as pltpu
from jax.experimental.pallas import tpu_sc as plsc

B = 4
N = 8192
S = 2048
R0, R1 = 0.2, 0.4
K0, K1 = 32, 32
CIN = 64
C2 = CIN * 2
C4 = C2 * 4

NC = 2
NS = 16
NW = NC * NS


def _fps_body(x_ref, y_ref, z_ref, ox_ref, oy_ref, oz_ref):
    X = x_ref[...]
    Y = y_ref[...]
    Z = z_ref[...]
    lin = (lax.broadcasted_iota(jnp.int32, (B, 64, 128), 1) * 128
           + lax.broadcasted_iota(jnp.int32, (B, 64, 128), 2))
    row8 = lax.broadcasted_iota(jnp.int32, (B, 8, 128), 1)
    lane = lax.broadcasted_iota(jnp.int32, (B, 8, 128), 2)
    BIG = jnp.int32(N)

    def pick(mask, V):
        m1 = jnp.max(jnp.where(mask, V, -jnp.inf), axis=2, keepdims=True)
        return jnp.max(m1, axis=1, keepdims=True)

    cx0 = X[:, 0:1, 0:1]
    cy0 = Y[:, 0:1, 0:1]
    cz0 = Z[:, 0:1, 0:1]
    D0 = jnp.full((B, 64, 128), 1e10, dtype=jnp.float32)
    acc0 = jnp.zeros((B, 8, 128), jnp.float32)

    def step(t, state):
        D, cx, cy, cz, ax, ay, az = state
        r = (t // 128) % 8
        l = t % 128
        cond = (row8 == r) & (lane == l)
        ax = jnp.where(cond, cx, ax)
        ay = jnp.where(cond, cy, ay)
        az = jnp.where(cond, cz, az)
        d = (X - cx) ** 2 + (Y - cy) ** 2 + (Z - cz) ** 2
        D = jnp.minimum(D, d)
        m1 = jnp.max(jnp.max(D, axis=2, keepdims=True), axis=1, keepdims=True)
        eq = D == m1
        sel = jnp.where(eq, lin, BIG)
        idx = jnp.min(jnp.min(sel, axis=2, keepdims=True), axis=1, keepdims=True)
        pickm = lin == idx
        ncx = pick(pickm, X)
        ncy = pick(pickm, Y)
        ncz = pick(pickm, Z)

        @pl.when(t % 1024 == 1023)
        def _():
            blk = t // 1024
            ox_ref[:, pl.ds(blk * 8, 8), :] = ax
            oy_ref[:, pl.ds(blk * 8, 8), :] = ay
            oz_ref[:, pl.ds(blk * 8, 8), :] = az

        return D, ncx, ncy, ncz, ax, ay, az

    lax.fori_loop(0, S, step, (D0, cx0, cy0, cz0, acc0, acc0, acc0))


def _fps_call(xb, yb, zb):
    o = jax.ShapeDtypeStruct((B, 16, 128), jnp.float32)
    return pl.pallas_call(_fps_body, out_shape=(o, o, o))(xb, yb, zb)


def _knn_body(qx_ref, qy_ref, qz_ref, px_ref, py_ref, pz_ref, o_ref, d_scr,
              *, n, k, r2, qt):
    b = pl.program_id(0)
    qx = qx_ref[0]
    qy = qy_ref[0]
    qz = qz_ref[0]
    CW = 128
    nch = n // CW
    INF = jnp.float32(jnp.inf)
    BIGF = jnp.float32(n)
    lane = lax.broadcasted_iota(jnp.int32, (qt, CW), 1).astype(jnp.float32)
    nm0 = jnp.full((qt, CW), INF)
    gi0 = jnp.full((qt, CW), BIGF)

    def build(c, carry):
        nm, gi = carry
        for sub in range(2):
            col = c * 2 * CW + sub * CW
            px = px_ref[0, :, pl.ds(col, CW)]
            py = py_ref[0, :, pl.ds(col, CW)]
            pz = pz_ref[0, :, pl.ds(col, CW)]
            d = (qx - px) ** 2 + (qy - py) ** 2 + (qz - pz) ** 2
            d_scr[:, pl.ds(col, CW)] = d
            gl = lane + col.astype(jnp.float32)
            better = d < nm
            nm = jnp.where(better, d, nm)
            gi = jnp.where(better, gl, gi)
        return nm, gi

    nm, gi = lax.fori_loop(0, nch // 2, build, (nm0, gi0))

    def epilogue(nm, gi):
        m = jnp.min(nm, axis=1, keepdims=True)
        idxf = jnp.min(jnp.where(nm == m, gi, BIGF), axis=1, keepdims=True)
        return m, idxf

    m, idx0 = epilogue(nm, gi)
    o_ref[0, :, pl.ds(0, 1)] = idx0.astype(jnp.int32) + b * n

    for j in range(1, k):
        def body(c, carry, m=m):
            nm, gi = carry
            for sub in range(2):
                col = c * 2 * CW + sub * CW
                dd = d_scr[:, pl.ds(col, CW)]
                gl = lane + col.astype(jnp.float32)
                cand = jnp.where(dd > m, dd, INF)
                better = cand < nm
                nm = jnp.where(better, cand, nm)
                gi = jnp.where(better, gl, gi)
            return nm, gi

        nm, gi = lax.fori_loop(0, nch // 2, body, (nm0, gi0))
        m, idxf = epilogue(nm, gi)
        oidx = jnp.where(m > r2, idx0, idxf)
        o_ref[0, :, pl.ds(j, 1)] = oidx.astype(jnp.int32) + b * n


def _knn_call(qx, qy, qz, px, py, pz, n, r2, k=32, qt=64):
    qspec = pl.BlockSpec((1, qt, 1), lambda b, s: (b, s, 0))
    pspec = pl.BlockSpec((1, 1, n), lambda b, s: (b, 0, 0))
    return pl.pallas_call(
        functools.partial(_knn_body, n=n, k=k, r2=r2, qt=qt),
        grid=(B, S // qt),
        in_specs=[qspec, qspec, qspec, pspec, pspec, pspec],
        out_specs=pl.BlockSpec((1, qt, k), lambda b, s: (b, s, 0)),
        out_shape=jax.ShapeDtypeStruct((B, S, k), jnp.int32),
        scratch_shapes=[pltpu.VMEM((qt, n), jnp.float32)],
    )(qx, qy, qz, px, py, pz)


def _dot(a, b):
    return lax.dot_general(a, b, (((1,), (0,)), ((), ())),
                           preferred_element_type=jnp.float32)


def _a1_body(g_ref, w_ref, o_ref):
    o_ref[0] = _dot(g_ref[0], w_ref[...])


def _a1_call(G1t, Wt):
    nb = 512
    return pl.pallas_call(
        _a1_body,
        grid=(B, N // nb),
        in_specs=[pl.BlockSpec((1, nb, CIN + 3), lambda b, i: (b, i, 0)),
                  pl.BlockSpec((CIN + 3, C2), lambda b, i: (0, 0))],
        out_specs=pl.BlockSpec((1, nb, C2), lambda b, i: (b, i, 0)),
        out_shape=jax.ShapeDtypeStruct((B, N, C2), jnp.float32),
    )(G1t, Wt)


def _mid_body(mg_ref, q_ref, wc1_ref, sab_ref, wf2_ref, wc2_ref,
              fea_ref, a2_ref, qc2_ref):
    q = q_ref[...]
    qc1 = _dot(q, wc1_ref[...])
    fea = jax.nn.relu(mg_ref[...] + sab_ref[...] - qc1)
    qc2 = _dot(q, wc2_ref[...])
    fea_ref[...] = fea
    a2_ref[...] = _dot(fea, wf2_ref[...]) + qc2
    qc2_ref[...] = qc2


def _mid_call(mg1, q3, wc1t, sab, wf2t, wc2t):
    nb = 512
    BS = B * S
    spec = pl.BlockSpec((nb, C2), lambda i: (i, 0))
    o = jax.ShapeDtypeStruct((BS, C2), jnp.float32)
    return pl.pallas_call(
        _mid_body,
        grid=(BS // nb,),
        in_specs=[spec,
                  pl.BlockSpec((nb, 3), lambda i: (i, 0)),
                  pl.BlockSpec((3, C2), lambda i: (0, 0)),
                  pl.BlockSpec((1, C2), lambda i: (0, 0)),
                  pl.BlockSpec((C2, C2), lambda i: (0, 0)),
                  pl.BlockSpec((3, C2), lambda i: (0, 0))],
        out_specs=[spec, spec, spec],
        out_shape=(o, o, o),
    )(mg1, q3, wc1t, sab, wf2t, wc2t)


def _final_body(mg_ref, qc2_ref, fea_ref, lab_ref, w1_ref, b1_ref, w2_ref,
                b2_ref, o_ref):
    fea2 = jax.nn.relu(mg_ref[...] + lab_ref[...] - qc2_ref[...])
    p = jax.nn.relu(_dot(fea2, w1_ref[...]) + b1_ref[...])
    p2 = _dot(p, w2_ref[...]) + b2_ref[...]
    o_ref[...] = jax.nn.relu(p2 + fea_ref[...])


def _final_call(mg2, qc2, fea, lab, w1t, b1, w2t, b2):
    nb = 512
    BS = B * S
    spec = pl.BlockSpec((nb, C2), lambda i: (i, 0))
    return pl.pallas_call(
        _final_body,
        grid=(BS // nb,),
        in_specs=[spec, spec, spec,
                  pl.BlockSpec((1, C2), lambda i: (0, 0)),
                  pl.BlockSpec((C2, C4), lambda i: (0, 0)),
                  pl.BlockSpec((1, C4), lambda i: (0, 0)),
                  pl.BlockSpec((C4, C2), lambda i: (0, 0)),
                  pl.BlockSpec((1, C2), lambda i: (0, 0))],
        out_specs=spec,
        out_shape=jax.ShapeDtypeStruct((BS, C2), jnp.float32),
    )(mg2, qc2, fea, lab, w1t, b1, w2t, b2)


def _gather_max(table, idx2d, k):
    Q = idx2d.shape[0] * 128 // k
    qpw = Q // NW
    QB = 128 // k
    nchunks = qpw // QB
    mesh = plsc.VectorSubcoreMesh(core_axis_name="c", subcore_axis_name="s",
                                  num_cores=NC, num_subcores=NS)

    @functools.partial(
        pl.kernel,
        out_type=jax.ShapeDtypeStruct((Q, C2), jnp.float32),
        mesh=mesh,
        scratch_types=[
            pltpu.VMEM((nchunks, 128), jnp.int32),
            pltpu.VMEM((128, C2), jnp.float32),
            pltpu.VMEM((128, C2), jnp.float32),
            pltpu.VMEM((qpw, C2), jnp.float32),
            pltpu.SemaphoreType.DMA,
            pltpu.SemaphoreType.DMA,
        ],
    )
    def gk(idx_hbm, table_hbm, out_hbm, idx_v, rows0, rows1, out_v, sem0, sem1):
        wid = lax.axis_index("s") * NC + lax.axis_index("c")
        pltpu.sync_copy(idx_hbm.at[pl.ds(wid * nchunks, nchunks)], idx_v)

        def start(i, rbuf, sem):
            pltpu.async_copy(table_hbm.at[idx_v.at[i]], rbuf, sem)

        def wait(i, rbuf, sem):
            pltpu.make_async_copy(table_hbm.at[idx_v.at[i]], rbuf, sem).wait()

        def compute(i, rbuf):
            def per_q(q, _):
                base = q * k
                acc = tuple(rbuf[base, c * 16:(c + 1) * 16] for c in range(8))

                def red(j, a):
                    return tuple(
                        jnp.maximum(a[c], rbuf[base + j, c * 16:(c + 1) * 16])
                        for c in range(8))

                acc = lax.fori_loop(1, k, red, acc)
                for c in range(8):
                    out_v[i * QB + q, c * 16:(c + 1) * 16] = acc[c]
                return 0

            lax.fori_loop(0, QB, per_q, 0)

        start(0, rows0, sem0)

        def pair(p, _):
            i0 = p * 2
            start(i0 + 1, rows1, sem1)
            wait(i0, rows0, sem0)
            compute(i0, rows0)

            @pl.when(i0 + 2 < nchunks)
            def _():
                start(i0 + 2, rows0, sem0)

            wait(i0 + 1, rows1, sem1)
            compute(i0 + 1, rows1)
            return 0

        lax.fori_loop(0, nchunks // 2, pair, 0)
        pltpu.sync_copy(out_v, out_hbm.at[pl.ds(wid * qpw, qpw)])

    return gk(idx2d, table)


def kernel(points_coor, points_fea, sa_W, sa_b, la_W, la_b,
           pw_W1, pw_b1, pw_W2, pw_b2):
    pc = points_coor
    x = pc[:, 0, :]
    y = pc[:, 1, :]
    z = pc[:, 2, :]
    ox, oy, oz = _fps_call(x.reshape(B, 64, 128), y.reshape(B, 64, 128),
                           z.reshape(B, 64, 128))
    qx = ox.reshape(B, S)
    qy = oy.reshape(B, S)
    qz = oz.reshape(B, S)

    idx1 = _knn_call(qx.reshape(B, S, 1), qy.reshape(B, S, 1),
                     qz.reshape(B, S, 1),
                     x.reshape(B, 1, N), y.reshape(B, 1, N),
                     z.reshape(B, 1, N), n=N, r2=R0 * R0, k=K0)
    G1t = jnp.concatenate([points_fea, pc / R0], axis=1).transpose(0, 2, 1)
    A1 = _a1_call(G1t, sa_W.T).reshape(B * N, C2)
    mg1 = _gather_max(A1, idx1.reshape(-1, 128), K0)

    q3 = jnp.stack([qx, qy, qz], axis=-1).reshape(B * S, 3)
    fea, A2, QC2 = _mid_call(mg1, q3, sa_W[:, CIN:].T / R0,
                             sa_b.reshape(1, C2), la_W[:, :C2].T,
                             la_W[:, C2:].T / R1)

    idx2 = _knn_call(qx.reshape(B, S, 1), qy.reshape(B, S, 1),
                     qz.reshape(B, S, 1),
                     qx.reshape(B, 1, S), qy.reshape(B, 1, S),
                     qz.reshape(B, 1, S), n=S, r2=R1 * R1, k=K1)
    mg2 = _gather_max(A2, idx2.reshape(-1, 128), K1)

    out = _final_call(mg2, QC2, fea, la_b.reshape(1, C2), pw_W1.T,
                      pw_b1.reshape(1, C4), pw_W2.T, pw_b2.reshape(1, C2))
    new_coor = jnp.stack([qx, qy, qz], axis=1)
    return new_coor, out.reshape(B, S, C2).transpose(0, 2, 1)


# speedup vs baseline: 10.1133x; 1.0002x over previous
import functools

import jax
import jax.numpy as jnp
from jax import lax
from jax.experimental import pallas as pl
from jax.experimental.pallas import tpu as pltpu
from jax.experimental.pallas import tpu_sc as plsc

B = 4
N = 8192
S = 2048
R0, R1 = 0.2, 0.4
K0, K1 = 32, 32
CIN = 64
C2 = CIN * 2
C4 = C2 * 4

NC = 2
NS = 16
NW = NC * NS


def _fps_body(x_ref, y_ref, z_ref, ox_ref, oy_ref, oz_ref):
    X = x_ref[...]
    Y = y_ref[...]
    Z = z_ref[...]
    lin = (lax.broadcasted_iota(jnp.int32, (B, 64, 128), 1) * 128
           + lax.broadcasted_iota(jnp.int32, (B, 64, 128), 2))
    row8 = lax.broadcasted_iota(jnp.int32, (B, 8, 128), 1)
    lane = lax.broadcasted_iota(jnp.int32, (B, 8, 128), 2)
    BIG = jnp.int32(N)

    def pick(mask, V):
        m1 = jnp.max(jnp.where(mask, V, -jnp.inf), axis=2, keepdims=True)
        return jnp.max(m1, axis=1, keepdims=True)

    cx0 = X[:, 0:1, 0:1]
    cy0 = Y[:, 0:1, 0:1]
    cz0 = Z[:, 0:1, 0:1]
    D0 = jnp.full((B, 64, 128), 1e10, dtype=jnp.float32)
    acc0 = jnp.zeros((B, 8, 128), jnp.float32)

    def step(t, state):
        D, cx, cy, cz, ax, ay, az = state
        r = (t // 128) % 8
        l = t % 128
        cond = (row8 == r) & (lane == l)
        ax = jnp.where(cond, cx, ax)
        ay = jnp.where(cond, cy, ay)
        az = jnp.where(cond, cz, az)
        d = (X - cx) ** 2 + (Y - cy) ** 2 + (Z - cz) ** 2
        D = jnp.minimum(D, d)
        m1 = jnp.max(jnp.max(D, axis=2, keepdims=True), axis=1, keepdims=True)
        eq = D == m1
        sel = jnp.where(eq, lin, BIG)
        idx = jnp.min(jnp.min(sel, axis=2, keepdims=True), axis=1, keepdims=True)
        pickm = lin == idx
        ncx = pick(pickm, X)
        ncy = pick(pickm, Y)
        ncz = pick(pickm, Z)

        @pl.when(t % 1024 == 1023)
        def _():
            blk = t // 1024
            ox_ref[:, pl.ds(blk * 8, 8), :] = ax
            oy_ref[:, pl.ds(blk * 8, 8), :] = ay
            oz_ref[:, pl.ds(blk * 8, 8), :] = az

        return D, ncx, ncy, ncz, ax, ay, az

    lax.fori_loop(0, S, step, (D0, cx0, cy0, cz0, acc0, acc0, acc0))


def _fps_call(xb, yb, zb):
    o = jax.ShapeDtypeStruct((B, 16, 128), jnp.float32)
    return pl.pallas_call(_fps_body, out_shape=(o, o, o))(xb, yb, zb)


def _knn_body(qx_ref, qy_ref, qz_ref, px_ref, py_ref, pz_ref, o_ref, d_scr,
              *, n, k, r2, qt):
    b = pl.program_id(0)
    qx = qx_ref[0]
    qy = qy_ref[0]
    qz = qz_ref[0]
    CW = 128
    nch = n // CW
    INF = jnp.float32(jnp.inf)
    BIGF = jnp.float32(n)
    lane = lax.broadcasted_iota(jnp.int32, (qt, CW), 1).astype(jnp.float32)
    nm0 = jnp.full((qt, CW), INF)
    gi0 = jnp.full((qt, CW), BIGF)

    def build(c, carry):
        nm, gi = carry
        for sub in range(2):
            col = c * 2 * CW + sub * CW
            px = px_ref[0, :, pl.ds(col, CW)]
            py = py_ref[0, :, pl.ds(col, CW)]
            pz = pz_ref[0, :, pl.ds(col, CW)]
            d = (qx - px) ** 2 + (qy - py) ** 2 + (qz - pz) ** 2
            d_scr[:, pl.ds(col, CW)] = d
            gl = lane + col.astype(jnp.float32)
            better = d < nm
            nm = jnp.where(better, d, nm)
            gi = jnp.where(better, gl, gi)
        return nm, gi

    nm, gi = lax.fori_loop(0, nch // 2, build, (nm0, gi0))

    def epilogue(nm, gi):
        m = jnp.min(nm, axis=1, keepdims=True)
        idxf = jnp.min(jnp.where(nm == m, gi, BIGF), axis=1, keepdims=True)
        return m, idxf

    m, idx0 = epilogue(nm, gi)
    o_ref[0, :, pl.ds(0, 1)] = idx0.astype(jnp.int32) + b * n

    for j in range(1, k):
        def body(c, carry, m=m):
            nm, gi = carry
            for sub in range(2):
                col = c * 2 * CW + sub * CW
                dd = d_scr[:, pl.ds(col, CW)]
                gl = lane + col.astype(jnp.float32)
                cand = jnp.where(dd > m, dd, INF)
                better = cand < nm
                nm = jnp.where(better, cand, nm)
                gi = jnp.where(better, gl, gi)
            return nm, gi

        nm, gi = lax.fori_loop(0, nch // 2, body, (nm0, gi0))
        m, idxf = epilogue(nm, gi)
        oidx = jnp.where(m > r2, idx0, idxf)
        o_ref[0, :, pl.ds(j, 1)] = oidx.astype(jnp.int32) + b * n


def _knn_call(qx, qy, qz, px, py, pz, n, r2, k=32, qt=64):
    qspec = pl.BlockSpec((1, qt, 1), lambda b, s: (b, s, 0))
    pspec = pl.BlockSpec((1, 1, n), lambda b, s: (b, 0, 0))
    return pl.pallas_call(
        functools.partial(_knn_body, n=n, k=k, r2=r2, qt=qt),
        grid=(B, S // qt),
        in_specs=[qspec, qspec, qspec, pspec, pspec, pspec],
        out_specs=pl.BlockSpec((1, qt, k), lambda b, s: (b, s, 0)),
        out_shape=jax.ShapeDtypeStruct((B, S, k), jnp.int32),
        scratch_shapes=[pltpu.VMEM((qt, n), jnp.float32)],
        compiler_params=pltpu.CompilerParams(
            dimension_semantics=("parallel", "parallel")),
    )(qx, qy, qz, px, py, pz)


def _dot(a, b):
    return lax.dot_general(a, b, (((1,), (0,)), ((), ())),
                           preferred_element_type=jnp.float32)


def _a1_body(g_ref, w_ref, o_ref):
    o_ref[0] = _dot(g_ref[0], w_ref[...])


def _a1_call(G1t, Wt):
    nb = 512
    return pl.pallas_call(
        _a1_body,
        grid=(B, N // nb),
        in_specs=[pl.BlockSpec((1, nb, CIN + 3), lambda b, i: (b, i, 0)),
                  pl.BlockSpec((CIN + 3, C2), lambda b, i: (0, 0))],
        out_specs=pl.BlockSpec((1, nb, C2), lambda b, i: (b, i, 0)),
        out_shape=jax.ShapeDtypeStruct((B, N, C2), jnp.float32),
    )(G1t, Wt)


def _mid_body(mg_ref, q_ref, wc1_ref, sab_ref, wf2_ref, wc2_ref,
              fea_ref, a2_ref, qc2_ref):
    q = q_ref[...]
    qc1 = _dot(q, wc1_ref[...])
    fea = jax.nn.relu(mg_ref[...] + sab_ref[...] - qc1)
    qc2 = _dot(q, wc2_ref[...])
    fea_ref[...] = fea
    a2_ref[...] = _dot(fea, wf2_ref[...]) + qc2
    qc2_ref[...] = qc2


def _mid_call(mg1, q3, wc1t, sab, wf2t, wc2t):
    nb = 512
    BS = B * S
    spec = pl.BlockSpec((nb, C2), lambda i: (i, 0))
    o = jax.ShapeDtypeStruct((BS, C2), jnp.float32)
    return pl.pallas_call(
        _mid_body,
        grid=(BS // nb,),
        in_specs=[spec,
                  pl.BlockSpec((nb, 3), lambda i: (i, 0)),
                  pl.BlockSpec((3, C2), lambda i: (0, 0)),
                  pl.BlockSpec((1, C2), lambda i: (0, 0)),
                  pl.BlockSpec((C2, C2), lambda i: (0, 0)),
                  pl.BlockSpec((3, C2), lambda i: (0, 0))],
        out_specs=[spec, spec, spec],
        out_shape=(o, o, o),
    )(mg1, q3, wc1t, sab, wf2t, wc2t)


def _final_body(mg_ref, qc2_ref, fea_ref, lab_ref, w1_ref, b1_ref, w2_ref,
                b2_ref, o_ref):
    fea2 = jax.nn.relu(mg_ref[...] + lab_ref[...] - qc2_ref[...])
    p = jax.nn.relu(_dot(fea2, w1_ref[...]) + b1_ref[...])
    p2 = _dot(p, w2_ref[...]) + b2_ref[...]
    o_ref[...] = jax.nn.relu(p2 + fea_ref[...])


def _final_call(mg2, qc2, fea, lab, w1t, b1, w2t, b2):
    nb = 512
    BS = B * S
    spec = pl.BlockSpec((nb, C2), lambda i: (i, 0))
    return pl.pallas_call(
        _final_body,
        grid=(BS // nb,),
        in_specs=[spec, spec, spec,
                  pl.BlockSpec((1, C2), lambda i: (0, 0)),
                  pl.BlockSpec((C2, C4), lambda i: (0, 0)),
                  pl.BlockSpec((1, C4), lambda i: (0, 0)),
                  pl.BlockSpec((C4, C2), lambda i: (0, 0)),
                  pl.BlockSpec((1, C2), lambda i: (0, 0))],
        out_specs=spec,
        out_shape=jax.ShapeDtypeStruct((BS, C2), jnp.float32),
    )(mg2, qc2, fea, lab, w1t, b1, w2t, b2)


def _gather_max(table, idx2d, k):
    Q = idx2d.shape[0] * 128 // k
    qpw = Q // NW
    QB = 128 // k
    nchunks = qpw // QB
    mesh = plsc.VectorSubcoreMesh(core_axis_name="c", subcore_axis_name="s",
                                  num_cores=NC, num_subcores=NS)

    @functools.partial(
        pl.kernel,
        out_type=jax.ShapeDtypeStruct((Q, C2), jnp.float32),
        mesh=mesh,
        scratch_types=[
            pltpu.VMEM((nchunks, 128), jnp.int32),
            pltpu.VMEM((128, C2), jnp.float32),
            pltpu.VMEM((128, C2), jnp.float32),
            pltpu.VMEM((qpw, C2), jnp.float32),
            pltpu.SemaphoreType.DMA,
            pltpu.SemaphoreType.DMA,
        ],
    )
    def gk(idx_hbm, table_hbm, out_hbm, idx_v, rows0, rows1, out_v, sem0, sem1):
        wid = lax.axis_index("s") * NC + lax.axis_index("c")
        pltpu.sync_copy(idx_hbm.at[pl.ds(wid * nchunks, nchunks)], idx_v)

        def start(i, rbuf, sem):
            pltpu.async_copy(table_hbm.at[idx_v.at[i]], rbuf, sem)

        def wait(i, rbuf, sem):
            pltpu.make_async_copy(table_hbm.at[idx_v.at[i]], rbuf, sem).wait()

        def compute(i, rbuf):
            def per_q(q, _):
                base = q * k
                acc = tuple(rbuf[base, c * 16:(c + 1) * 16] for c in range(8))

                def red(j, a):
                    return tuple(
                        jnp.maximum(a[c], rbuf[base + j, c * 16:(c + 1) * 16])
                        for c in range(8))

                acc = lax.fori_loop(1, k, red, acc)
                for c in range(8):
                    out_v[i * QB + q, c * 16:(c + 1) * 16] = acc[c]
                return 0

            lax.fori_loop(0, QB, per_q, 0)

        start(0, rows0, sem0)

        def pair(p, _):
            i0 = p * 2
            start(i0 + 1, rows1, sem1)
            wait(i0, rows0, sem0)
            compute(i0, rows0)

            @pl.when(i0 + 2 < nchunks)
            def _():
                start(i0 + 2, rows0, sem0)

            wait(i0 + 1, rows1, sem1)
            compute(i0 + 1, rows1)
            return 0

        lax.fori_loop(0, nchunks // 2, pair, 0)
        pltpu.sync_copy(out_v, out_hbm.at[pl.ds(wid * qpw, qpw)])

    return gk(idx2d, table)


def kernel(points_coor, points_fea, sa_W, sa_b, la_W, la_b,
           pw_W1, pw_b1, pw_W2, pw_b2):
    pc = points_coor
    x = pc[:, 0, :]
    y = pc[:, 1, :]
    z = pc[:, 2, :]
    ox, oy, oz = _fps_call(x.reshape(B, 64, 128), y.reshape(B, 64, 128),
                           z.reshape(B, 64, 128))
    qx = ox.reshape(B, S)
    qy = oy.reshape(B, S)
    qz = oz.reshape(B, S)

    idx1 = _knn_call(qx.reshape(B, S, 1), qy.reshape(B, S, 1),
                     qz.reshape(B, S, 1),
                     x.reshape(B, 1, N), y.reshape(B, 1, N),
                     z.reshape(B, 1, N), n=N, r2=R0 * R0, k=K0)
    G1t = jnp.concatenate([points_fea, pc / R0], axis=1).transpose(0, 2, 1)
    A1 = _a1_call(G1t, sa_W.T).reshape(B * N, C2)
    mg1 = _gather_max(A1, idx1.reshape(-1, 128), K0)

    q3 = jnp.stack([qx, qy, qz], axis=-1).reshape(B * S, 3)
    fea, A2, QC2 = _mid_call(mg1, q3, sa_W[:, CIN:].T / R0,
                             sa_b.reshape(1, C2), la_W[:, :C2].T,
                             la_W[:, C2:].T / R1)

    idx2 = _knn_call(qx.reshape(B, S, 1), qy.reshape(B, S, 1),
                     qz.reshape(B, S, 1),
                     qx.reshape(B, 1, S), qy.reshape(B, 1, S),
                     qz.reshape(B, 1, S), n=S, r2=R1 * R1, k=K1)
    mg2 = _gather_max(A2, idx2.reshape(-1, 128), K1)

    out = _final_call(mg2, QC2, fea, la_b.reshape(1, C2), pw_W1.T,
                      pw_b1.reshape(1, C4), pw_W2.T, pw_b2.reshape(1, C2))
    new_coor = jnp.stack([qx, qy, qz], axis=1)
    return new_coor, out.reshape(B, S, C2).transpose(0, 2, 1)


# speedup vs baseline: 10.1246x; 1.0011x over previous
import functools

import jax
import jax.numpy as jnp
from jax import lax
from jax.experimental import pallas as pl
from jax.experimental.pallas import tpu as pltpu
from jax.experimental.pallas import tpu_sc as plsc

B = 4
N = 8192
S = 2048
R0, R1 = 0.2, 0.4
K0, K1 = 32, 32
CIN = 64
C2 = CIN * 2
C4 = C2 * 4

NC = 2
NS = 16
NW = NC * NS


def _fps_body(x_ref, y_ref, z_ref, ox_ref, oy_ref, oz_ref):
    lin = (lax.broadcasted_iota(jnp.int32, (B, 64, 128), 1) * 128
           + lax.broadcasted_iota(jnp.int32, (B, 64, 128), 2))
    row8 = lax.broadcasted_iota(jnp.int32, (B, 8, 128), 1)
    lane = lax.broadcasted_iota(jnp.int32, (B, 8, 128), 2)
    BIG = jnp.int32(N)

    def pick(mask, ref):
        m1 = jnp.max(jnp.where(mask, ref[...], -jnp.inf), axis=2,
                     keepdims=True)
        return jnp.max(m1, axis=1, keepdims=True)

    cx0 = x_ref[:, 0:1, 0:1]
    cy0 = y_ref[:, 0:1, 0:1]
    cz0 = z_ref[:, 0:1, 0:1]
    D0 = jnp.full((B, 64, 128), 1e10, dtype=jnp.float32)
    acc0 = jnp.zeros((B, 8, 128), jnp.float32)

    def step(t, state):
        D, cx, cy, cz, ax, ay, az = state
        r = (t // 128) % 8
        l = t % 128
        cond = (row8 == r) & (lane == l)
        ax = jnp.where(cond, cx, ax)
        ay = jnp.where(cond, cy, ay)
        az = jnp.where(cond, cz, az)
        d = ((x_ref[...] - cx) ** 2 + (y_ref[...] - cy) ** 2
             + (z_ref[...] - cz) ** 2)
        D = jnp.minimum(D, d)
        m1 = jnp.max(jnp.max(D, axis=2, keepdims=True), axis=1, keepdims=True)
        eq = D == m1
        sel = jnp.where(eq, lin, BIG)
        idx = jnp.min(jnp.min(sel, axis=2, keepdims=True), axis=1, keepdims=True)
        pickm = lin == idx
        ncx = pick(pickm, x_ref)
        ncy = pick(pickm, y_ref)
        ncz = pick(pickm, z_ref)

        @pl.when(t % 1024 == 1023)
        def _():
            blk = t // 1024
            ox_ref[:, pl.ds(blk * 8, 8), :] = ax
            oy_ref[:, pl.ds(blk * 8, 8), :] = ay
            oz_ref[:, pl.ds(blk * 8, 8), :] = az

        return D, ncx, ncy, ncz, ax, ay, az

    lax.fori_loop(0, S, step, (D0, cx0, cy0, cz0, acc0, acc0, acc0))


def _fps_call(xb, yb, zb):
    o = jax.ShapeDtypeStruct((B, 16, 128), jnp.float32)
    return pl.pallas_call(_fps_body, out_shape=(o, o, o))(xb, yb, zb)


def _knn_body(qx_ref, qy_ref, qz_ref, px_ref, py_ref, pz_ref, o_ref, d_scr,
              *, n, k, r2, qt):
    b = pl.program_id(0)
    qx = qx_ref[0]
    qy = qy_ref[0]
    qz = qz_ref[0]
    CW = 128
    nch = n // CW
    INF = jnp.float32(jnp.inf)
    BIGF = jnp.float32(n)
    lane = lax.broadcasted_iota(jnp.int32, (qt, CW), 1).astype(jnp.float32)
    nm0 = jnp.full((qt, CW), INF)
    gi0 = jnp.full((qt, CW), BIGF)

    def build(c, carry):
        nm, gi = carry
        for sub in range(2):
            col = c * 2 * CW + sub * CW
            px = px_ref[0, :, pl.ds(col, CW)]
            py = py_ref[0, :, pl.ds(col, CW)]
            pz = pz_ref[0, :, pl.ds(col, CW)]
            d = (qx - px) ** 2 + (qy - py) ** 2 + (qz - pz) ** 2
            d_scr[:, pl.ds(col, CW)] = d
            gl = lane + col.astype(jnp.float32)
            better = d < nm
            nm = jnp.where(better, d, nm)
            gi = jnp.where(better, gl, gi)
        return nm, gi

    nm, gi = lax.fori_loop(0, nch // 2, build, (nm0, gi0))

    def epilogue(nm, gi):
        m = jnp.min(nm, axis=1, keepdims=True)
        idxf = jnp.min(jnp.where(nm == m, gi, BIGF), axis=1, keepdims=True)
        return m, idxf

    m, idx0 = epilogue(nm, gi)
    o_ref[0, :, pl.ds(0, 1)] = idx0.astype(jnp.int32) + b * n

    for j in range(1, k):
        def body(c, carry, m=m):
            nm, gi = carry
            for sub in range(2):
                col = c * 2 * CW + sub * CW
                dd = d_scr[:, pl.ds(col, CW)]
                gl = lane + col.astype(jnp.float32)
                cand = jnp.where(dd > m, dd, INF)
                better = cand < nm
                nm = jnp.where(better, cand, nm)
                gi = jnp.where(better, gl, gi)
            return nm, gi

        nm, gi = lax.fori_loop(0, nch // 2, body, (nm0, gi0))
        m, idxf = epilogue(nm, gi)
        oidx = jnp.where(m > r2, idx0, idxf)
        o_ref[0, :, pl.ds(j, 1)] = oidx.astype(jnp.int32) + b * n


def _knn_call(qx, qy, qz, px, py, pz, n, r2, k=32, qt=64):
    qspec = pl.BlockSpec((1, qt, 1), lambda b, s: (b, s, 0))
    pspec = pl.BlockSpec((1, 1, n), lambda b, s: (b, 0, 0))
    return pl.pallas_call(
        functools.partial(_knn_body, n=n, k=k, r2=r2, qt=qt),
        grid=(B, S // qt),
        in_specs=[qspec, qspec, qspec, pspec, pspec, pspec],
        out_specs=pl.BlockSpec((1, qt, k), lambda b, s: (b, s, 0)),
        out_shape=jax.ShapeDtypeStruct((B, S, k), jnp.int32),
        scratch_shapes=[pltpu.VMEM((qt, n), jnp.float32)],
        compiler_params=pltpu.CompilerParams(
            dimension_semantics=("parallel", "parallel")),
    )(qx, qy, qz, px, py, pz)


def _dot(a, b):
    return lax.dot_general(a, b, (((1,), (0,)), ((), ())),
                           preferred_element_type=jnp.float32)


def _a1_body(g_ref, w_ref, o_ref):
    o_ref[0] = _dot(g_ref[0], w_ref[...])


def _a1_call(G1t, Wt):
    nb = 512
    return pl.pallas_call(
        _a1_body,
        grid=(B, N // nb),
        in_specs=[pl.BlockSpec((1, nb, CIN + 3), lambda b, i: (b, i, 0)),
                  pl.BlockSpec((CIN + 3, C2), lambda b, i: (0, 0))],
        out_specs=pl.BlockSpec((1, nb, C2), lambda b, i: (b, i, 0)),
        out_shape=jax.ShapeDtypeStruct((B, N, C2), jnp.float32),
    )(G1t, Wt)


def _mid_body(mg_ref, q_ref, wc1_ref, sab_ref, wf2_ref, wc2_ref,
              fea_ref, a2_ref, qc2_ref):
    q = q_ref[...]
    qc1 = _dot(q, wc1_ref[...])
    fea = jax.nn.relu(mg_ref[...] + sab_ref[...] - qc1)
    qc2 = _dot(q, wc2_ref[...])
    fea_ref[...] = fea
    a2_ref[...] = _dot(fea, wf2_ref[...]) + qc2
    qc2_ref[...] = qc2


def _mid_call(mg1, q3, wc1t, sab, wf2t, wc2t):
    nb = 512
    BS = B * S
    spec = pl.BlockSpec((nb, C2), lambda i: (i, 0))
    o = jax.ShapeDtypeStruct((BS, C2), jnp.float32)
    return pl.pallas_call(
        _mid_body,
        grid=(BS // nb,),
        in_specs=[spec,
                  pl.BlockSpec((nb, 3), lambda i: (i, 0)),
                  pl.BlockSpec((3, C2), lambda i: (0, 0)),
                  pl.BlockSpec((1, C2), lambda i: (0, 0)),
                  pl.BlockSpec((C2, C2), lambda i: (0, 0)),
                  pl.BlockSpec((3, C2), lambda i: (0, 0))],
        out_specs=[spec, spec, spec],
        out_shape=(o, o, o),
    )(mg1, q3, wc1t, sab, wf2t, wc2t)


def _final_body(mg_ref, qc2_ref, fea_ref, lab_ref, w1_ref, b1_ref, w2_ref,
                b2_ref, o_ref):
    fea2 = jax.nn.relu(mg_ref[...] + lab_ref[...] - qc2_ref[...])
    p = jax.nn.relu(_dot(fea2, w1_ref[...]) + b1_ref[...])
    p2 = _dot(p, w2_ref[...]) + b2_ref[...]
    o_ref[...] = jax.nn.relu(p2 + fea_ref[...])


def _final_call(mg2, qc2, fea, lab, w1t, b1, w2t, b2):
    nb = 512
    BS = B * S
    spec = pl.BlockSpec((nb, C2), lambda i: (i, 0))
    return pl.pallas_call(
        _final_body,
        grid=(BS // nb,),
        in_specs=[spec, spec, spec,
                  pl.BlockSpec((1, C2), lambda i: (0, 0)),
                  pl.BlockSpec((C2, C4), lambda i: (0, 0)),
                  pl.BlockSpec((1, C4), lambda i: (0, 0)),
                  pl.BlockSpec((C4, C2), lambda i: (0, 0)),
                  pl.BlockSpec((1, C2), lambda i: (0, 0))],
        out_specs=spec,
        out_shape=jax.ShapeDtypeStruct((BS, C2), jnp.float32),
    )(mg2, qc2, fea, lab, w1t, b1, w2t, b2)


def _gather_max(table, idx2d, k):
    Q = idx2d.shape[0] * 128 // k
    qpw = Q // NW
    QB = 128 // k
    nchunks = qpw // QB
    mesh = plsc.VectorSubcoreMesh(core_axis_name="c", subcore_axis_name="s",
                                  num_cores=NC, num_subcores=NS)

    @functools.partial(
        pl.kernel,
        out_type=jax.ShapeDtypeStruct((Q, C2), jnp.float32),
        mesh=mesh,
        scratch_types=[
            pltpu.VMEM((nchunks, 128), jnp.int32),
            pltpu.VMEM((128, C2), jnp.float32),
            pltpu.VMEM((128, C2), jnp.float32),
            pltpu.VMEM((qpw, C2), jnp.float32),
            pltpu.SemaphoreType.DMA,
            pltpu.SemaphoreType.DMA,
        ],
    )
    def gk(idx_hbm, table_hbm, out_hbm, idx_v, rows0, rows1, out_v, sem0, sem1):
        wid = lax.axis_index("s") * NC + lax.axis_index("c")
        pltpu.sync_copy(idx_hbm.at[pl.ds(wid * nchunks, nchunks)], idx_v)

        def start(i, rbuf, sem):
            pltpu.async_copy(table_hbm.at[idx_v.at[i]], rbuf, sem)

        def wait(i, rbuf, sem):
            pltpu.make_async_copy(table_hbm.at[idx_v.at[i]], rbuf, sem).wait()

        def compute(i, rbuf):
            def per_q(q, _):
                base = q * k
                acc = tuple(rbuf[base, c * 16:(c + 1) * 16] for c in range(8))

                def red(j, a):
                    return tuple(
                        jnp.maximum(a[c], rbuf[base + j, c * 16:(c + 1) * 16])
                        for c in range(8))

                acc = lax.fori_loop(1, k, red, acc)
                for c in range(8):
                    out_v[i * QB + q, c * 16:(c + 1) * 16] = acc[c]
                return 0

            lax.fori_loop(0, QB, per_q, 0)

        start(0, rows0, sem0)

        def pair(p, _):
            i0 = p * 2
            start(i0 + 1, rows1, sem1)
            wait(i0, rows0, sem0)
            compute(i0, rows0)

            @pl.when(i0 + 2 < nchunks)
            def _():
                start(i0 + 2, rows0, sem0)

            wait(i0 + 1, rows1, sem1)
            compute(i0 + 1, rows1)
            return 0

        lax.fori_loop(0, nchunks // 2, pair, 0)
        pltpu.sync_copy(out_v, out_hbm.at[pl.ds(wid * qpw, qpw)])

    return gk(idx2d, table)


def kernel(points_coor, points_fea, sa_W, sa_b, la_W, la_b,
           pw_W1, pw_b1, pw_W2, pw_b2):
    pc = points_coor
    x = pc[:, 0, :]
    y = pc[:, 1, :]
    z = pc[:, 2, :]
    ox, oy, oz = _fps_call(x.reshape(B, 64, 128), y.reshape(B, 64, 128),
                           z.reshape(B, 64, 128))
    qx = ox.reshape(B, S)
    qy = oy.reshape(B, S)
    qz = oz.reshape(B, S)

    idx1 = _knn_call(qx.reshape(B, S, 1), qy.reshape(B, S, 1),
                     qz.reshape(B, S, 1),
                     x.reshape(B, 1, N), y.reshape(B, 1, N),
                     z.reshape(B, 1, N), n=N, r2=R0 * R0, k=K0)
    G1t = jnp.concatenate([points_fea, pc / R0], axis=1).transpose(0, 2, 1)
    A1 = _a1_call(G1t, sa_W.T).reshape(B * N, C2)
    mg1 = _gather_max(A1, idx1.reshape(-1, 128), K0)

    q3 = jnp.stack([qx, qy, qz], axis=-1).reshape(B * S, 3)
    fea, A2, QC2 = _mid_call(mg1, q3, sa_W[:, CIN:].T / R0,
                             sa_b.reshape(1, C2), la_W[:, :C2].T,
                             la_W[:, C2:].T / R1)

    idx2 = _knn_call(qx.reshape(B, S, 1), qy.reshape(B, S, 1),
                     qz.reshape(B, S, 1),
                     qx.reshape(B, 1, S), qy.reshape(B, 1, S),
                     qz.reshape(B, 1, S), n=S, r2=R1 * R1, k=K1)
    mg2 = _gather_max(A2, idx2.reshape(-1, 128), K1)

    out = _final_call(mg2, QC2, fea, la_b.reshape(1, C2), pw_W1.T,
                      pw_b1.reshape(1, C4), pw_W2.T, pw_b2.reshape(1, C2))
    new_coor = jnp.stack([qx, qy, qz], axis=1)
    return new_coor, out.reshape(B, S, C2).transpose(0, 2, 1)


# speedup vs baseline: 11.3382x; 1.1199x over previous
import functools

import jax
import jax.numpy as jnp
from jax import lax
from jax.experimental import pallas as pl
from jax.experimental.pallas import tpu as pltpu
from jax.experimental.pallas import tpu_sc as plsc

B = 4
N = 8192
S = 2048
R0, R1 = 0.2, 0.4
K0, K1 = 32, 32
CIN = 64
C2 = CIN * 2
C4 = C2 * 4

NC = 2
NS = 16
NW = NC * NS


def _fps_body(x_ref, y_ref, z_ref, ox_ref, oy_ref, oz_ref):
    lin = (lax.broadcasted_iota(jnp.int32, (B, 64, 128), 1) * 128
           + lax.broadcasted_iota(jnp.int32, (B, 64, 128), 2))
    row8 = lax.broadcasted_iota(jnp.int32, (B, 8, 128), 1)
    lane = lax.broadcasted_iota(jnp.int32, (B, 8, 128), 2)
    BIG = jnp.int32(N)

    def pick(mask, ref):
        m1 = jnp.max(jnp.where(mask, ref[...], -jnp.inf), axis=2,
                     keepdims=True)
        return jnp.max(m1, axis=1, keepdims=True)

    cx0 = x_ref[:, 0:1, 0:1]
    cy0 = y_ref[:, 0:1, 0:1]
    cz0 = z_ref[:, 0:1, 0:1]
    D0 = jnp.full((B, 64, 128), 1e10, dtype=jnp.float32)
    acc0 = jnp.zeros((B, 8, 128), jnp.float32)

    def step(t, state):
        D, cx, cy, cz, ax, ay, az = state
        r = (t // 128) % 8
        l = t % 128
        cond = (row8 == r) & (lane == l)
        ax = jnp.where(cond, cx, ax)
        ay = jnp.where(cond, cy, ay)
        az = jnp.where(cond, cz, az)
        d = ((x_ref[...] - cx) ** 2 + (y_ref[...] - cy) ** 2
             + (z_ref[...] - cz) ** 2)
        D = jnp.minimum(D, d)
        m1 = jnp.max(jnp.max(D, axis=2, keepdims=True), axis=1, keepdims=True)
        eq = D == m1
        sel = jnp.where(eq, lin, BIG)
        idx = jnp.min(jnp.min(sel, axis=2, keepdims=True), axis=1, keepdims=True)
        pickm = lin == idx
        ncx = pick(pickm, x_ref)
        ncy = pick(pickm, y_ref)
        ncz = pick(pickm, z_ref)

        @pl.when(t % 1024 == 1023)
        def _():
            blk = t // 1024
            ox_ref[:, pl.ds(blk * 8, 8), :] = ax
            oy_ref[:, pl.ds(blk * 8, 8), :] = ay
            oz_ref[:, pl.ds(blk * 8, 8), :] = az

        return D, ncx, ncy, ncz, ax, ay, az

    lax.fori_loop(0, S, step, (D0, cx0, cy0, cz0, acc0, acc0, acc0))


def _fps_call(xb, yb, zb):
    o = jax.ShapeDtypeStruct((B, 16, 128), jnp.float32)
    return pl.pallas_call(_fps_body, out_shape=(o, o, o))(xb, yb, zb)


def _knn_body(qx_ref, qy_ref, qz_ref, px_ref, py_ref, pz_ref, o_ref, d_scr,
              *, n, k, r2, qt):
    b = pl.program_id(0)
    qx = qx_ref[0]
    qy = qy_ref[0]
    qz = qz_ref[0]
    CW = 128
    nch = n // CW
    INF = jnp.float32(jnp.inf)
    BIGF = jnp.float32(n)
    lane = lax.broadcasted_iota(jnp.int32, (qt, CW), 1).astype(jnp.float32)
    nm0 = jnp.full((qt, CW), INF)
    gi0 = jnp.full((qt, CW), BIGF)

    def build(c, carry):
        nm, gi = carry
        for sub in range(4):
            col = c * 4 * CW + sub * CW
            px = px_ref[0, :, pl.ds(col, CW)]
            py = py_ref[0, :, pl.ds(col, CW)]
            pz = pz_ref[0, :, pl.ds(col, CW)]
            d = (qx - px) ** 2 + (qy - py) ** 2 + (qz - pz) ** 2
            d_scr[:, pl.ds(col, CW)] = d
            gl = lane + col.astype(jnp.float32)
            better = d < nm
            nm = jnp.where(better, d, nm)
            gi = jnp.where(better, gl, gi)
        return nm, gi

    nm, gi = lax.fori_loop(0, nch // 4, build, (nm0, gi0))

    def epilogue(nm, gi):
        m = jnp.min(nm, axis=1, keepdims=True)
        idxf = jnp.min(jnp.where(nm == m, gi, BIGF), axis=1, keepdims=True)
        return m, idxf

    m, idx0 = epilogue(nm, gi)
    o_ref[0, :, pl.ds(0, 1)] = idx0.astype(jnp.int32) + b * n

    for j in range(1, k):
        def body(c, carry, m=m):
            nm, gi = carry
            for sub in range(4):
                col = c * 4 * CW + sub * CW
                dd = d_scr[:, pl.ds(col, CW)]
                gl = lane + col.astype(jnp.float32)
                cand = jnp.where(dd > m, dd, INF)
                better = cand < nm
                nm = jnp.where(better, cand, nm)
                gi = jnp.where(better, gl, gi)
            return nm, gi

        nm, gi = lax.fori_loop(0, nch // 4, body, (nm0, gi0))
        m, idxf = epilogue(nm, gi)
        oidx = jnp.where(m > r2, idx0, idxf)
        o_ref[0, :, pl.ds(j, 1)] = oidx.astype(jnp.int32) + b * n


def _knn_call(qx, qy, qz, px, py, pz, n, r2, k=32, qt=64):
    qspec = pl.BlockSpec((1, qt, 1), lambda b, s: (b, s, 0))
    pspec = pl.BlockSpec((1, 1, n), lambda b, s: (b, 0, 0))
    return pl.pallas_call(
        functools.partial(_knn_body, n=n, k=k, r2=r2, qt=qt),
        grid=(B, S // qt),
        in_specs=[qspec, qspec, qspec, pspec, pspec, pspec],
        out_specs=pl.BlockSpec((1, qt, k), lambda b, s: (b, s, 0)),
        out_shape=jax.ShapeDtypeStruct((B, S, k), jnp.int32),
        scratch_shapes=[pltpu.VMEM((qt, n), jnp.float32)],
        compiler_params=pltpu.CompilerParams(
            dimension_semantics=("parallel", "parallel")),
    )(qx, qy, qz, px, py, pz)


def _dot(a, b):
    return lax.dot_general(a, b, (((1,), (0,)), ((), ())),
                           preferred_element_type=jnp.float32)


def _a1_body(g_ref, w_ref, o_ref):
    o_ref[0] = _dot(g_ref[0], w_ref[...])


def _a1_call(G1t, Wt):
    nb = 512
    return pl.pallas_call(
        _a1_body,
        grid=(B, N // nb),
        in_specs=[pl.BlockSpec((1, nb, CIN + 3), lambda b, i: (b, i, 0)),
                  pl.BlockSpec((CIN + 3, C2), lambda b, i: (0, 0))],
        out_specs=pl.BlockSpec((1, nb, C2), lambda b, i: (b, i, 0)),
        out_shape=jax.ShapeDtypeStruct((B, N, C2), jnp.float32),
    )(G1t, Wt)


def _mid_body(mg_ref, q_ref, wc1_ref, sab_ref, wf2_ref, wc2_ref,
              fea_ref, a2_ref, qc2_ref):
    q = q_ref[...]
    qc1 = _dot(q, wc1_ref[...])
    fea = jax.nn.relu(mg_ref[...] + sab_ref[...] - qc1)
    qc2 = _dot(q, wc2_ref[...])
    fea_ref[...] = fea
    a2_ref[...] = _dot(fea, wf2_ref[...]) + qc2
    qc2_ref[...] = qc2


def _mid_call(mg1, q3, wc1t, sab, wf2t, wc2t):
    nb = 512
    BS = B * S
    spec = pl.BlockSpec((nb, C2), lambda i: (i, 0))
    o = jax.ShapeDtypeStruct((BS, C2), jnp.float32)
    return pl.pallas_call(
        _mid_body,
        grid=(BS // nb,),
        in_specs=[spec,
                  pl.BlockSpec((nb, 3), lambda i: (i, 0)),
                  pl.BlockSpec((3, C2), lambda i: (0, 0)),
                  pl.BlockSpec((1, C2), lambda i: (0, 0)),
                  pl.BlockSpec((C2, C2), lambda i: (0, 0)),
                  pl.BlockSpec((3, C2), lambda i: (0, 0))],
        out_specs=[spec, spec, spec],
        out_shape=(o, o, o),
    )(mg1, q3, wc1t, sab, wf2t, wc2t)


def _final_body(mg_ref, qc2_ref, fea_ref, lab_ref, w1_ref, b1_ref, w2_ref,
                b2_ref, o_ref):
    fea2 = jax.nn.relu(mg_ref[...] + lab_ref[...] - qc2_ref[...])
    p = jax.nn.relu(_dot(fea2, w1_ref[...]) + b1_ref[...])
    p2 = _dot(p, w2_ref[...]) + b2_ref[...]
    o_ref[...] = jax.nn.relu(p2 + fea_ref[...])


def _final_call(mg2, qc2, fea, lab, w1t, b1, w2t, b2):
    nb = 512
    BS = B * S
    spec = pl.BlockSpec((nb, C2), lambda i: (i, 0))
    return pl.pallas_call(
        _final_body,
        grid=(BS // nb,),
        in_specs=[spec, spec, spec,
                  pl.BlockSpec((1, C2), lambda i: (0, 0)),
                  pl.BlockSpec((C2, C4), lambda i: (0, 0)),
                  pl.BlockSpec((1, C4), lambda i: (0, 0)),
                  pl.BlockSpec((C4, C2), lambda i: (0, 0)),
                  pl.BlockSpec((1, C2), lambda i: (0, 0))],
        out_specs=spec,
        out_shape=jax.ShapeDtypeStruct((BS, C2), jnp.float32),
    )(mg2, qc2, fea, lab, w1t, b1, w2t, b2)


def _gather_max(table, idx2d, k):
    Q = idx2d.shape[0] * 128 // k
    qpw = Q // NW
    QB = 128 // k
    nchunks = qpw // QB
    mesh = plsc.VectorSubcoreMesh(core_axis_name="c", subcore_axis_name="s",
                                  num_cores=NC, num_subcores=NS)

    @functools.partial(
        pl.kernel,
        out_type=jax.ShapeDtypeStruct((Q, C2), jnp.float32),
        mesh=mesh,
        scratch_types=[
            pltpu.VMEM((nchunks, 128), jnp.int32),
            pltpu.VMEM((128, C2), jnp.float32),
            pltpu.VMEM((128, C2), jnp.float32),
            pltpu.VMEM((qpw, C2), jnp.float32),
            pltpu.SemaphoreType.DMA,
            pltpu.SemaphoreType.DMA,
        ],
    )
    def gk(idx_hbm, table_hbm, out_hbm, idx_v, rows0, rows1, out_v, sem0, sem1):
        wid = lax.axis_index("s") * NC + lax.axis_index("c")
        pltpu.sync_copy(idx_hbm.at[pl.ds(wid * nchunks, nchunks)], idx_v)

        def start(i, rbuf, sem):
            pltpu.async_copy(table_hbm.at[idx_v.at[i]], rbuf, sem)

        def wait(i, rbuf, sem):
            pltpu.make_async_copy(table_hbm.at[idx_v.at[i]], rbuf, sem).wait()

        def compute(i, rbuf):
            def per_q(q, _):
                base = q * k
                acc = tuple(rbuf[base, c * 16:(c + 1) * 16] for c in range(8))

                def red(j, a):
                    return tuple(
                        jnp.maximum(a[c], rbuf[base + j, c * 16:(c + 1) * 16])
                        for c in range(8))

                acc = lax.fori_loop(1, k, red, acc)
                for c in range(8):
                    out_v[i * QB + q, c * 16:(c + 1) * 16] = acc[c]
                return 0

            lax.fori_loop(0, QB, per_q, 0)

        start(0, rows0, sem0)

        def pair(p, _):
            i0 = p * 2
            start(i0 + 1, rows1, sem1)
            wait(i0, rows0, sem0)
            compute(i0, rows0)

            @pl.when(i0 + 2 < nchunks)
            def _():
                start(i0 + 2, rows0, sem0)

            wait(i0 + 1, rows1, sem1)
            compute(i0 + 1, rows1)
            return 0

        lax.fori_loop(0, nchunks // 2, pair, 0)
        pltpu.sync_copy(out_v, out_hbm.at[pl.ds(wid * qpw, qpw)])

    return gk(idx2d, table)


def kernel(points_coor, points_fea, sa_W, sa_b, la_W, la_b,
           pw_W1, pw_b1, pw_W2, pw_b2):
    pc = points_coor
    x = pc[:, 0, :]
    y = pc[:, 1, :]
    z = pc[:, 2, :]
    ox, oy, oz = _fps_call(x.reshape(B, 64, 128), y.reshape(B, 64, 128),
                           z.reshape(B, 64, 128))
    qx = ox.reshape(B, S)
    qy = oy.reshape(B, S)
    qz = oz.reshape(B, S)

    idx1 = _knn_call(qx.reshape(B, S, 1), qy.reshape(B, S, 1),
                     qz.reshape(B, S, 1),
                     x.reshape(B, 1, N), y.reshape(B, 1, N),
                     z.reshape(B, 1, N), n=N, r2=R0 * R0, k=K0)
    G1t = jnp.concatenate([points_fea, pc / R0], axis=1).transpose(0, 2, 1)
    A1 = _a1_call(G1t, sa_W.T).reshape(B * N, C2)
    mg1 = _gather_max(A1, idx1.reshape(-1, 128), K0)

    q3 = jnp.stack([qx, qy, qz], axis=-1).reshape(B * S, 3)
    fea, A2, QC2 = _mid_call(mg1, q3, sa_W[:, CIN:].T / R0,
                             sa_b.reshape(1, C2), la_W[:, :C2].T,
                             la_W[:, C2:].T / R1)

    idx2 = _knn_call(qx.reshape(B, S, 1), qy.reshape(B, S, 1),
                     qz.reshape(B, S, 1),
                     qx.reshape(B, 1, S), qy.reshape(B, 1, S),
                     qz.reshape(B, 1, S), n=S, r2=R1 * R1, k=K1)
    mg2 = _gather_max(A2, idx2.reshape(-1, 128), K1)

    out = _final_call(mg2, QC2, fea, la_b.reshape(1, C2), pw_W1.T,
                      pw_b1.reshape(1, C4), pw_W2.T, pw_b2.reshape(1, C2))
    new_coor = jnp.stack([qx, qy, qz], axis=1)
    return new_coor, out.reshape(B, S, C2).transpose(0, 2, 1)
